# baseline (speedup 1.0000x reference)
import functools

import jax
import jax.numpy as jnp
from jax import lax
from jax.experimental import pallas as pl
from jax.experimental.pallas import tpu as pltpu
from jax.experimental.pallas import tpu_sc as plsc

N = 10000; E = 160000; M = 64; K = 16; RREL = 2; D = 128; HEADS = 8; HID = 16
SEMD = 8; PC = 8; EMG = 512
ROW = D + 2 * HEADS
GROW = ROW + 16
NC, NS = 2, 16
NW = NC * NS
EPT = E // NW
CH = 40
NCHUNK = EPT // CH
NBUF = 3
RPT = N // NS
BLK = 1000
NBLK = N // BLK

_LEAK = 0.2


def _leaky(x):
    return jnp.where(x > 0, x, _LEAK * x)


@functools.cache
def _sc_gat_kernel():
  mesh = plsc.VectorSubcoreMesh(core_axis_name="c", subcore_axis_name="s")

  @functools.partial(
    pl.kernel,
    out_type=jax.ShapeDtypeStruct((NC, N, ROW), jnp.float32),
    mesh=mesh,
    scratch_types=[
        pltpu.VMEM((NBUF, CH), jnp.int32),
        pltpu.VMEM((NBUF, CH), jnp.int32),
        pltpu.VMEM((NBUF * CH, ROW), jnp.float32),
        pltpu.VMEM((NBUF * CH, 16), jnp.float32),
        pltpu.VMEM_SHARED((N, ROW), jnp.float32),
        pltpu.SemaphoreType.DMA,
        pltpu.SemaphoreType.DMA,
        pltpu.SemaphoreType.DMA,
        pltpu.SemaphoreType.DMA,
        pltpu.SemaphoreType.DMA,
        pltpu.SemaphoreType.DMA,
        pltpu.SemaphoreType.DMA,
        pltpu.SemaphoreType.DMA,
        pltpu.SemaphoreType.DMA,
    ],
    compiler_params=pltpu.CompilerParams(use_tc_tiling_on_sc=False),
  )
  def body(hs_hbm, sd_hbm, src_hbm, dst_hbm, out_hbm,
           src_v, dst_v, rows_v, sd_v, acc, *sems):
    semh = sems[0:NBUF]
    semd = sems[NBUF:2 * NBUF]
    semc = sems[2 * NBUF:3 * NBUF]
    cid = lax.axis_index("c")
    sid = lax.axis_index("s")
    wid = cid * NS + sid

    def _zrow(i, _):
        for j in range(ROW // 16):
            rows_v[i, pl.ds(16 * j, 16)] = jnp.zeros((16,), jnp.float32)
        return 0
    lax.fori_loop(0, NBUF * CH, _zrow, 0)
    ZB = NBUF * CH
    for b in range(RPT // ZB):
        pltpu.sync_copy(rows_v, acc.at[pl.ds(sid * RPT + b * ZB, ZB)])
    rem = RPT % ZB
    if rem:
        pltpu.sync_copy(rows_v.at[pl.ds(0, rem)],
                        acc.at[pl.ds(sid * RPT + (RPT // ZB) * ZB, rem)])
    plsc.subcore_barrier()

    lanes = lax.iota(jnp.int32, 16)
    head_mask = lanes < HEADS

    def _prefetch(g, b):
        base = wid * EPT + g * CH
        pltpu.sync_copy(src_hbm.at[pl.ds(base, CH)], src_v.at[b])
        pltpu.sync_copy(dst_hbm.at[pl.ds(base, CH)], dst_v.at[b])
        pltpu.async_copy(hs_hbm.at[src_v.at[b]],
                         rows_v.at[pl.ds(b * CH, CH)], semh[b])
        pltpu.async_copy(sd_hbm.at[dst_v.at[b]],
                         sd_v.at[pl.ds(b * CH, CH)], semd[b])

    def _wait_rows(b):
        pltpu.make_async_copy(hs_hbm.at[src_v.at[b]],
                              rows_v.at[pl.ds(b * CH, CH)], semh[b]).wait()
        pltpu.make_async_copy(sd_hbm.at[dst_v.at[b]],
                              sd_v.at[pl.ds(b * CH, CH)], semd[b]).wait()

    def _compute(b):
        def _edge(i, _):
            k = b * CH + i
            ssrc = rows_v[k, pl.ds(D, 16)]
            sdst = sd_v[k]
            e = ssrc + sdst
            e = jnp.where(e > 0, e, _LEAK * e)
            ex = jnp.where(head_mask, jnp.exp(e), 0.0)
            rows_v[k, pl.ds(D, 16)] = ex
            for j in range(HEADS):
                a = jnp.full((16,), ex[j], jnp.float32)
                rows_v[k, pl.ds(16 * j, 16)] = rows_v[k, pl.ds(16 * j, 16)] * a
            return 0
        lax.fori_loop(0, CH, _edge, 0, unroll=4)

    def _scatter_start(b):
        pltpu.async_copy(rows_v.at[pl.ds(b * CH, CH)],
                         acc.at[dst_v.at[b]], semc[b], add=True)

    def _scatter_wait(b):
        pltpu.make_async_copy(rows_v.at[pl.ds(b * CH, CH)],
                              acc.at[dst_v.at[b]], semc[b]).wait()

    _prefetch(0, 0)
    _prefetch(1, 1)

    def _triple(t, _):
        for j in range(NBUF):
            g = NBUF * t + j
            _wait_rows(j)
            _compute(j)
            b2 = (j + 2) % NBUF

            @pl.when(g >= 1)
            def _():
                _scatter_wait(b2)
            _prefetch(g + 2, b2)
            _scatter_start(j)
        return 0
    lax.fori_loop(0, (NCHUNK - 2) // NBUF, _triple, 0)

    for g, j in ((NCHUNK - 2, 0), (NCHUNK - 1, 1)):
        _wait_rows(j)
        _compute(j)
        _scatter_wait((j + 2) % NBUF)
        _scatter_start(j)
    _scatter_wait(1)

    plsc.subcore_barrier()
    pltpu.sync_copy(acc.at[pl.ds(sid * RPT, RPT)],
                    out_hbm.at[cid, pl.ds(sid * RPT, RPT)])

  return body


def _sc_gat(hs, sd, src, dst):
    return _sc_gat_kernel()(hs, sd, src, dst)


_GPT = (M * K) // NW


@functools.cache
def _sc_gather_kernel():
  mesh = plsc.VectorSubcoreMesh(core_axis_name="c", subcore_axis_name="s")

  @functools.partial(
    pl.kernel,
    out_type=jax.ShapeDtypeStruct((M * K, D), jnp.float32),
    mesh=mesh,
    scratch_types=[
        pltpu.VMEM((_GPT,), jnp.int32),
        pltpu.VMEM((_GPT, D), jnp.float32),
        pltpu.SemaphoreType.DMA,
    ],
    compiler_params=pltpu.CompilerParams(use_tc_tiling_on_sc=False),
  )
  def body(x_hbm, idx_hbm, out_hbm, idx_v, rows_v, sem):
    cid = lax.axis_index("c")
    sid = lax.axis_index("s")
    wid = cid * NS + sid
    base = wid * _GPT
    pltpu.sync_copy(idx_hbm.at[pl.ds(base, _GPT)], idx_v)
    pltpu.async_copy(x_hbm.at[idx_v], rows_v, sem).wait()
    pltpu.sync_copy(rows_v, out_hbm.at[pl.ds(base, _GPT)])

  return body


def _sc_gather(x, idx):
    return _sc_gather_kernel()(x, idx)


def _mm_body(ngat, with_affine, *refs):
    x_ref, w_ref, as_ref, ad_ref = refs[:4]
    nin = 4 + (2 if with_affine else 0)
    out_refs = refs[nin:]
    x = x_ref[...]
    if with_affine:
        x = x * refs[4][0:1, :] + refs[5][0:1, :]
    y = jnp.dot(x, w_ref[...], preferred_element_type=jnp.float32)
    zero8 = jnp.zeros((BLK, HEADS), jnp.float32)
    for g in range(ngat):
        h = y[:, g * D:(g + 1) * D]
        hr = h.reshape(BLK, HEADS, HID)
        ssrc = jnp.sum(hr * as_ref[...][None, g * HEADS:(g + 1) * HEADS, :], -1)
        sdst = jnp.sum(hr * ad_ref[...][None, g * HEADS:(g + 1) * HEADS, :], -1)
        out_refs[2 * g][...] = jnp.concatenate([h, ssrc, zero8], 1)
        out_refs[2 * g + 1][...] = jnp.concatenate([sdst, zero8], 1)


def _mm_proj(x, w, a_s, a_d, ngat, scale=None, shift=None):
    P = ngat * D
    with_affine = scale is not None
    outs = []
    out_specs = []
    for _ in range(ngat):
        outs.append(jax.ShapeDtypeStruct((N, ROW), jnp.float32))
        outs.append(jax.ShapeDtypeStruct((N, 16), jnp.float32))
        out_specs.append(pl.BlockSpec((BLK, ROW), lambda i: (i, 0)))
        out_specs.append(pl.BlockSpec((BLK, 16), lambda i: (i, 0)))
    in_specs = [pl.BlockSpec((BLK, D), lambda i: (i, 0)),
                pl.BlockSpec((D, P), lambda i: (0, 0)),
                pl.BlockSpec((ngat * HEADS, HID), lambda i: (0, 0)),
                pl.BlockSpec((ngat * HEADS, HID), lambda i: (0, 0))]
    ins = [x, w, a_s, a_d]
    if with_affine:
        in_specs += [pl.BlockSpec((8, D), lambda i: (0, 0)),
                     pl.BlockSpec((8, D), lambda i: (0, 0))]
        ins += [_pad8(scale), _pad8(shift)]
    return pl.pallas_call(
        functools.partial(_mm_body, ngat, with_affine),
        grid=(NBLK,),
        in_specs=in_specs,
        out_specs=out_specs,
        out_shape=tuple(outs),
    )(*ins)


def _norm_acc(a):
    s = a[0] + a[1]
    den = s[:, D:D + HEADS]
    z = s[:, :D].reshape(-1, HEADS, HID) / (den[:, :, None] + 1e-16)
    return z.reshape(-1, D)


def _post_body(nz, with_extra, *refs):
    wq_ref = refs[nz]
    bq_ref = refs[nz + 1]
    qv_ref = refs[nz + 2]
    nin = nz + 3 + (1 if with_extra else 0)
    zouts = refs[nin:nin + nz]
    wp_ref = refs[nin + nz + (1 if with_extra else 0)]
    cols = jnp.zeros((8, D), jnp.float32)
    row0 = lax.broadcasted_iota(jnp.int32, (8, D), 0) == 0
    iota = lax.broadcasted_iota(jnp.int32, (8, D), 1)
    for r in range(nz):
        z = _norm_acc(refs[r][...])
        zouts[r][...] = z
        t = jnp.tanh(jnp.dot(z, wq_ref[...],
                             preferred_element_type=jnp.float32) + bq_ref[0:1, :])
        w = jnp.sum(t * qv_ref[0:1, :], axis=1)
        cols = cols + jnp.where(row0 & (iota == r), jnp.sum(w), 0.0)
    if with_extra:
        refs[nin + nz][...] = _norm_acc(refs[nz + 3][...])
    wp_ref[...] = cols


def _post(accs, extra_acc, wq, bq, qv):
    nz = len(accs)
    with_extra = extra_acc is not None
    ins = list(accs) + [wq, bq, qv] + ([extra_acc] if with_extra else [])
    in_specs = ([pl.BlockSpec((NC, BLK, ROW), lambda i: (0, i, 0))] * nz
                + [pl.BlockSpec((D, SEMD), lambda i: (0, 0)),
                   pl.BlockSpec((8, SEMD), lambda i: (0, 0)),
                   pl.BlockSpec((8, SEMD), lambda i: (0, 0))]
                + ([pl.BlockSpec((NC, BLK, ROW), lambda i: (0, i, 0))]
                   if with_extra else []))
    outs = ([jax.ShapeDtypeStruct((N, D), jnp.float32)] * nz
            + ([jax.ShapeDtypeStruct((N, D), jnp.float32)] if with_extra else [])
            + [jax.ShapeDtypeStruct((NBLK * 8, D), jnp.float32)])
    out_specs = ([pl.BlockSpec((BLK, D), lambda i: (i, 0))] * nz
                 + ([pl.BlockSpec((BLK, D), lambda i: (i, 0))] if with_extra else [])
                 + [pl.BlockSpec((8, D), lambda i: (i, 0))])
    res = pl.pallas_call(
        functools.partial(_post_body, nz, with_extra),
        grid=(NBLK,),
        in_specs=in_specs,
        out_specs=out_specs,
        out_shape=tuple(outs),
    )(*ins)
    return res


def _combine_body(z0_ref, z1_ref, ex_ref, beta_ref, sem_ref, out_ref):
    s = beta_ref[0] * z0_ref[...] + beta_ref[1] * z1_ref[...]
    sem_ref[...] = s
    out_ref[...] = _leaky(s + ex_ref[...])


def _combine(z0, z1, extra, beta):
    return pl.pallas_call(
        _combine_body,
        grid=(NBLK,),
        in_specs=[pl.BlockSpec((BLK, D), lambda i: (i, 0)),
                  pl.BlockSpec((BLK, D), lambda i: (i, 0)),
                  pl.BlockSpec((BLK, D), lambda i: (i, 0)),
                  pl.BlockSpec(memory_space=pltpu.SMEM)],
        out_specs=[pl.BlockSpec((BLK, D), lambda i: (i, 0)),
                   pl.BlockSpec((BLK, D), lambda i: (i, 0))],
        out_shape=(jax.ShapeDtypeStruct((N, D), jnp.float32),
                   jax.ShapeDtypeStruct((N, D), jnp.float32)),
    )(z0, z1, extra, beta)


def _contrib_body(tn_ref, attn_ref, msg_ref, out_ref):
    i = pl.program_id(0)
    colid = i * BLK + lax.broadcasted_iota(jnp.int32, (M, BLK), 1)
    wt = jnp.zeros((M, BLK), jnp.float32)
    for k in range(K):
        hit = (tn_ref[:, k][:, None] == colid).astype(jnp.float32)
        wt = wt + attn_ref[:, k][:, None] * hit
    out_ref[...] = lax.dot_general(wt, msg_ref[...], (((0,), (0,)), ((), ())),
                                   preferred_element_type=jnp.float32)


def _contrib(tn, attn_pad, msg):
    return pl.pallas_call(
        _contrib_body,
        grid=(NBLK,),
        in_specs=[pl.BlockSpec((M, K), lambda i: (0, 0)),
                  pl.BlockSpec((M, D), lambda i: (0, 0)),
                  pl.BlockSpec((M, D), lambda i: (0, 0))],
        out_specs=pl.BlockSpec((BLK, D), lambda i: (i, 0)),
        out_shape=jax.ShapeDtypeStruct((N, D), jnp.float32),
    )(tn, attn_pad, msg)


def _lnbn_body(x_ref, g_ref, b_ref, y_ref, s1_ref, s2_ref):
    x = x_ref[...]
    mu = jnp.mean(x, -1, keepdims=True)
    v = jnp.mean(x * x, -1, keepdims=True) - mu * mu
    y = _leaky(g_ref[0:1, :] * (x - mu) * lax.rsqrt(v + 1e-5) + b_ref[0:1, :])
    y_ref[...] = y
    row0 = lax.broadcasted_iota(jnp.int32, (8, D), 0) == 0
    s1_ref[...] = jnp.where(row0, jnp.sum(y, axis=0, keepdims=True), 0.0)
    s2_ref[...] = jnp.where(row0, jnp.sum(y * y, axis=0, keepdims=True), 0.0)


def _lnbn_pass(x, g8, b8):
    return pl.pallas_call(
        _lnbn_body,
        grid=(NBLK,),
        in_specs=[pl.BlockSpec((BLK, D), lambda i: (i, 0)),
                  pl.BlockSpec((8, D), lambda i: (0, 0)),
                  pl.BlockSpec((8, D), lambda i: (0, 0))],
        out_specs=[pl.BlockSpec((BLK, D), lambda i: (i, 0)),
                   pl.BlockSpec((8, D), lambda i: (i, 0)),
                   pl.BlockSpec((8, D), lambda i: (i, 0))],
        out_shape=(jax.ShapeDtypeStruct((N, D), jnp.float32),
                   jax.ShapeDtypeStruct((NBLK * 8, D), jnp.float32),
                   jax.ShapeDtypeStruct((NBLK * 8, D), jnp.float32)),
    )(x, g8, b8)


def _dense_gat(h, ssrc, sdst, src, dst, n):
    oh_dst_n = (lax.broadcasted_iota(jnp.int32, (n, EMG), 0)
                == dst[None, :]).astype(jnp.float32)
    oh_src_e = (lax.broadcasted_iota(jnp.int32, (EMG, n), 1)
                == src[:, None]).astype(jnp.float32)
    oh_dst_e = (lax.broadcasted_iota(jnp.int32, (EMG, n), 1)
                == dst[:, None]).astype(jnp.float32)
    sc_src = jnp.dot(oh_src_e, ssrc, preferred_element_type=jnp.float32)
    sc_dst = jnp.dot(oh_dst_e, sdst, preferred_element_type=jnp.float32)
    e = _leaky(sc_src + sc_dst)
    big = jnp.float32(-1e30)
    m = jnp.max(jnp.where(oh_dst_n[:, :, None] > 0, e[None, :, :], big), axis=1)
    m = jnp.where(m <= big * 0.5, 0.0, m)
    ex = jnp.exp(e - jnp.dot(oh_dst_e, m, preferred_element_type=jnp.float32))
    den = jnp.dot(oh_dst_n, ex, preferred_element_type=jnp.float32)
    alpha = ex / (jnp.dot(oh_dst_e, den, preferred_element_type=jnp.float32) + 1e-16)
    hsrc = jnp.dot(oh_src_e, h, preferred_element_type=jnp.float32)
    wrow = (alpha[:, :, None] * hsrc.reshape(EMG, HEADS, HID)).reshape(EMG, D)
    return jnp.dot(oh_dst_n, wrow, preferred_element_type=jnp.float32)


def _mg_body(do_norm, *refs):
    (mgx_ref, nb_ref, mp_ref, ei_ref,
     wm_ref, bsm_ref, bdm_ref, wsem_ref, bq_ref, qv_ref,
     wpc_ref, vpc_ref, wpool_ref, wme_ref, bsme_ref, bdme_ref, wcci_ref,
     lng_ref, lnb_ref, bng_ref, bnb_ref,
     mg_out_ref, attn_out_ref, msg_ref) = refs
    x = mgx_ref[...]
    if do_norm:
        mu = jnp.mean(x, -1, keepdims=True)
        v = jnp.mean(x * x, -1, keepdims=True) - mu * mu
        x = _leaky(lng_ref[0:1, :] * (x - mu) * lax.rsqrt(v + 1e-5) + lnb_ref[0:1, :])
        mu2 = jnp.mean(x, 0, keepdims=True)
        v2 = jnp.mean(x * x, 0, keepdims=True) - mu2 * mu2
        x = bng_ref[0:1, :] * (x - mu2) * lax.rsqrt(v2 + 1e-5) + bnb_ref[0:1, :]

    mp = mp_ref[...].reshape(RREL * 2 * EMG)
    ei = ei_ref[...].reshape(2 * EMG)

    zs = []
    ws = []
    for r in range(RREL):
        w = wm_ref[...][r * D:(r + 1) * D, :]
        h = jnp.dot(x, w, preferred_element_type=jnp.float32)
        ssrc = jnp.dot(h, bsm_ref[...][r * D:(r + 1) * D, :],
                       preferred_element_type=jnp.float32)
        sdst = jnp.dot(h, bdm_ref[...][r * D:(r + 1) * D, :],
                       preferred_element_type=jnp.float32)
        src = mp[r * 2 * EMG:r * 2 * EMG + EMG]
        dst = mp[r * 2 * EMG + EMG:(r + 1) * 2 * EMG]
        z = _dense_gat(h, ssrc, sdst, src, dst, M)
        zs.append(z)
        t = jnp.tanh(jnp.dot(z, wsem_ref[...],
                             preferred_element_type=jnp.float32) + bq_ref[0:1, :])
        ws.append(jnp.mean(jnp.sum(t * qv_ref[0:1, :], axis=1)))
    w0 = ws[0]; w1 = ws[1]
    mx = jnp.maximum(w0, w1)
    e0 = jnp.exp(w0 - mx); e1 = jnp.exp(w1 - mx)
    b0 = e0 / (e0 + e1); b1 = e1 / (e0 + e1)
    mg_sem = b0 * zs[0] + b1 * zs[1]

    nb = nb_ref[...]
    t = jnp.tanh(jnp.dot(nb, wpc_ref[...], preferred_element_type=jnp.float32))
    y = jnp.sum(t * vpc_ref[0:1, :], axis=1).reshape(M, K)
    ymax = jnp.max(y, axis=1, keepdims=True)
    yex = jnp.exp(y - ymax)
    attn = yex / jnp.sum(yex, axis=1, keepdims=True)
    pooled = jnp.sum(attn[:, :, None] * nb.reshape(M, K, D), axis=1)

    he = jnp.dot(x, wme_ref[...], preferred_element_type=jnp.float32)
    ssrc = jnp.dot(he, bsme_ref[...], preferred_element_type=jnp.float32)
    sdst = jnp.dot(he, bdme_ref[...], preferred_element_type=jnp.float32)
    src = ei[0:EMG]
    dst = ei[EMG:2 * EMG]
    ge = _dense_gat(he, ssrc, sdst, src, dst, M)

    mg = _leaky(mg_sem + jnp.dot(pooled, wpool_ref[...],
                                 preferred_element_type=jnp.float32) + ge)
    mg_out_ref[...] = mg
    attn_out_ref[...] = jnp.concatenate(
        [attn, jnp.zeros((M, D - K), jnp.float32)], axis=1)
    msg_ref[...] = jnp.dot(mg, wcci_ref[...], preferred_element_type=jnp.float32)


def _mg_all(do_norm, mgx, nb, mp_i, ei_i, wm, bsm, bdm, wsem, bq, qv,
            wpc, vpc, wpool, wme, bsme, bdme, wcci, lng, lnb, bng, bnb):
    full = lambda s: pl.BlockSpec(s, lambda: tuple(0 for _ in s))
    ins = [mgx, nb, mp_i, ei_i, wm, bsm, bdm, wsem, bq, qv,
           wpc, vpc, wpool, wme, bsme, bdme, wcci, lng, lnb, bng, bnb]
    in_specs = [full(tuple(a.shape)) for a in ins]
    return pl.pallas_call(
        functools.partial(_mg_body, do_norm),
        in_specs=in_specs,
        out_specs=[full((M, D)), full((M, D)), full((M, D))],
        out_shape=(jax.ShapeDtypeStruct((M, D), jnp.float32),
                   jax.ShapeDtypeStruct((M, D), jnp.float32),
                   jax.ShapeDtypeStruct((M, D), jnp.float32)),
    )(*ins)


def _blockdiag(a):
    eye = jnp.eye(HEADS, dtype=a.dtype)
    return (eye[:, None, :] * a[:, :, None]).reshape(D, HEADS)


def _pad8(v):
    return jnp.broadcast_to(v[None, :], (8, v.shape[0]))


def _beta_from_partials(wp, n):
    t = jnp.sum(wp, axis=0)
    w = t[:2] / n
    return jax.nn.softmax(w)


def _up_layer(x, mgx, mp_srcdst, ei_srcdst, mg_mp_i, mg_ei_i, tn_flat, p,
              wcci, scale=None, shift=None, do_mg_norm=False, ln=None):
    wcat = jnp.concatenate([p['Wp'][0], p['Wp'][1], p['Wpe']], 1)
    a_s = jnp.concatenate([p['asp'][0], p['asp'][1], p['aspe']], 0)
    a_d = jnp.concatenate([p['adp'][0], p['adp'][1], p['adpe']], 0)
    hs0, sd0, hs1, sd1, hs2, sd2 = _mm_proj(x, wcat, a_s, a_d, 3,
                                            scale=scale, shift=shift)

    acc0 = _sc_gat(hs0, sd0, mp_srcdst[0][0], mp_srcdst[0][1])
    acc1 = _sc_gat(hs1, sd1, mp_srcdst[1][0], mp_srcdst[1][1])
    acc2 = _sc_gat(hs2, sd2, ei_srcdst[0], ei_srcdst[1])

    z0, z1, ze, wp = _post([acc0, acc1], acc2,
                           p['Wsem_p'], _pad8(p['bsem_p']), _pad8(p['qsem_p']))
    beta = _beta_from_partials(wp, N)
    ppi_sem, ppi_out = _combine(z0, z1, ze, beta)

    nb = _sc_gather(ppi_sem, tn_flat)
    lng, lnb, bng, bnb = ln
    mg_out, attn_pad, msg = _mg_all(
        do_mg_norm, mgx, nb, mg_mp_i, mg_ei_i,
        p['Wm'].reshape(RREL * D, D),
        jnp.concatenate([_blockdiag(p['asm'][r]) for r in range(RREL)], 0),
        jnp.concatenate([_blockdiag(p['adm'][r]) for r in range(RREL)], 0),
        p['Wsem_m'], _pad8(p['bsem_m']), _pad8(p['qsem_m']),
        p['Wpc'], _pad8(p['vpc']), p['Wpool'],
        p['Wme'], _blockdiag(p['asme']), _blockdiag(p['adme']),
        wcci,
        _pad8(lng), _pad8(lnb), _pad8(bng), _pad8(bnb))
    return ppi_out, mg_out, attn_pad, msg


def _down_layer(x, msg, attn_pad, tn, mp_srcdst, p):
    wcat = jnp.concatenate([p['Wd'][0], p['Wd'][1]], 1)
    a_s = jnp.concatenate([p['asd'][0], p['asd'][1]], 0)
    a_d = jnp.concatenate([p['add'][0], p['add'][1]], 0)
    hs0, sd0, hs1, sd1 = _mm_proj(x, wcat, a_s, a_d, 2)

    acc0 = _sc_gat(hs0, sd0, mp_srcdst[0][0], mp_srcdst[0][1])
    acc1 = _sc_gat(hs1, sd1, mp_srcdst[1][0], mp_srcdst[1][1])

    z0, z1, wp = _post([acc0, acc1], None,
                       p['Wsem_d'], _pad8(p['bsem_d']), _pad8(p['qsem_d']))
    beta = _beta_from_partials(wp, N)
    contrib = _contrib(tn, attn_pad, msg)
    _, out = _combine(z0, z1, contrib, beta)
    return out


def kernel(ppi_x, mg_x, ppi_metapaths, mg_metapaths, ppi_edge_index,
           mg_edge_index, tissue_neighbors, params):
    mp = [(ppi_metapaths[r, 0], ppi_metapaths[r, 1]) for r in range(RREL)]
    ei = (ppi_edge_index[0], ppi_edge_index[1])
    mg_mp_i = mg_metapaths.reshape(RREL * 2 * EMG // D, D)
    mg_ei_i = mg_edge_index.reshape(2 * EMG // D, D)
    tn_flat = tissue_neighbors.reshape(M * K)
    ln = (params['ln_g'], params['ln_b'], params['bn_g'], params['bn_b'])

    p1u, p1d = params['conv1_up'], params['conv1_down']
    p2u, p2d = params['conv2_up'], params['conv2_down']

    ppi1, mg1, attn1, msg1 = _up_layer(ppi_x, mg_x, mp, ei, mg_mp_i, mg_ei_i,
                                       tn_flat, p1u, p1d['Wcci'], ln=ln)
    ppi2 = _down_layer(ppi1, msg1, attn1, tissue_neighbors, mp, p1d)

    y, s1, s2 = _lnbn_pass(ppi2, _pad8(params['ln_g']), _pad8(params['ln_b']))
    mu = jnp.sum(s1, 0) / N
    var = jnp.sum(s2, 0) / N - mu * mu
    scale = params['bn_g'] * lax.rsqrt(var + 1e-5)
    shift = params['bn_b'] - mu * scale

    ppi3, mg2, attn2, msg2 = _up_layer(y, mg1, mp, ei, mg_mp_i, mg_ei_i,
                                       tn_flat, p2u, p2d['Wcci'],
                                       scale=scale, shift=shift,
                                       do_mg_norm=True, ln=ln)
    ppi4 = _down_layer(ppi3, msg2, attn2, tissue_neighbors, mp, p2d)
    return ppi4, mg2


# speedup vs baseline: 1.0100x; 1.0100x over previous
import functools

import jax
import jax.numpy as jnp
from jax import lax
from jax.experimental import pallas as pl
from jax.experimental.pallas import tpu as pltpu
from jax.experimental.pallas import tpu_sc as plsc

N = 10000; E = 160000; M = 64; K = 16; RREL = 2; D = 128; HEADS = 8; HID = 16
SEMD = 8; PC = 8; EMG = 512
ROW = D + 2 * HEADS
GROW = ROW + 16
NC, NS = 2, 16
NW = NC * NS
EPT = E // NW
CH = 40
NCHUNK = EPT // CH
NBUF = 3
RPT = N // NS
BLK = 1000
NBLK = N // BLK

_LEAK = 0.2


def _leaky(x):
    return jnp.where(x > 0, x, _LEAK * x)


@functools.cache
def _sc_gat_kernel():
  mesh = plsc.VectorSubcoreMesh(core_axis_name="c", subcore_axis_name="s")

  @functools.partial(
    pl.kernel,
    out_type=jax.ShapeDtypeStruct((NC, N, ROW), jnp.float32),
    mesh=mesh,
    scratch_types=[
        pltpu.VMEM((NBUF, CH), jnp.int32),
        pltpu.VMEM((NBUF, CH), jnp.int32),
        pltpu.VMEM((NBUF * CH, ROW), jnp.float32),
        pltpu.VMEM((NBUF * CH, 16), jnp.float32),
        pltpu.VMEM_SHARED((N, ROW), jnp.float32),
        pltpu.SemaphoreType.DMA,
        pltpu.SemaphoreType.DMA,
        pltpu.SemaphoreType.DMA,
        pltpu.SemaphoreType.DMA,
        pltpu.SemaphoreType.DMA,
        pltpu.SemaphoreType.DMA,
        pltpu.SemaphoreType.DMA,
        pltpu.SemaphoreType.DMA,
        pltpu.SemaphoreType.DMA,
    ],
    compiler_params=pltpu.CompilerParams(use_tc_tiling_on_sc=False),
  )
  def body(hs_hbm, sd_hbm, src_hbm, dst_hbm, out_hbm,
           src_v, dst_v, rows_v, sd_v, acc, *sems):
    semh = sems[0:NBUF]
    semd = sems[NBUF:2 * NBUF]
    semc = sems[2 * NBUF:3 * NBUF]
    cid = lax.axis_index("c")
    sid = lax.axis_index("s")
    wid = cid * NS + sid

    def _zrow(i, _):
        for j in range(ROW // 16):
            rows_v[i, pl.ds(16 * j, 16)] = jnp.zeros((16,), jnp.float32)
        return 0
    lax.fori_loop(0, NBUF * CH, _zrow, 0)
    ZB = NBUF * CH
    for b in range(RPT // ZB):
        pltpu.sync_copy(rows_v, acc.at[pl.ds(sid * RPT + b * ZB, ZB)])
    rem = RPT % ZB
    if rem:
        pltpu.sync_copy(rows_v.at[pl.ds(0, rem)],
                        acc.at[pl.ds(sid * RPT + (RPT // ZB) * ZB, rem)])
    plsc.subcore_barrier()

    lanes = lax.iota(jnp.int32, 16)
    head_mask = lanes < HEADS

    def _prefetch(g, b):
        base = wid * EPT + g * CH
        pltpu.sync_copy(src_hbm.at[pl.ds(base, CH)], src_v.at[b])
        pltpu.sync_copy(dst_hbm.at[pl.ds(base, CH)], dst_v.at[b])
        pltpu.async_copy(hs_hbm.at[src_v.at[b]],
                         rows_v.at[pl.ds(b * CH, CH)], semh[b])
        pltpu.async_copy(sd_hbm.at[dst_v.at[b]],
                         sd_v.at[pl.ds(b * CH, CH)], semd[b])

    def _wait_rows(b):
        pltpu.make_async_copy(hs_hbm.at[src_v.at[b]],
                              rows_v.at[pl.ds(b * CH, CH)], semh[b]).wait()
        pltpu.make_async_copy(sd_hbm.at[dst_v.at[b]],
                              sd_v.at[pl.ds(b * CH, CH)], semd[b]).wait()

    def _compute(b):
        def _edge(i, _):
            k = b * CH + i
            ssrc = rows_v[k, pl.ds(D, 16)]
            sdst = sd_v[k]
            e = ssrc + sdst
            e = jnp.where(e > 0, e, _LEAK * e)
            ex = jnp.where(head_mask, jnp.exp(e), 0.0)
            rows_v[k, pl.ds(D, 16)] = ex
            for j in range(HEADS):
                a = jnp.full((16,), ex[j], jnp.float32)
                rows_v[k, pl.ds(16 * j, 16)] = rows_v[k, pl.ds(16 * j, 16)] * a
            return 0
        lax.fori_loop(0, CH, _edge, 0, unroll=2)

    def _scatter_start(b):
        pltpu.async_copy(rows_v.at[pl.ds(b * CH, CH)],
                         acc.at[dst_v.at[b]], semc[b], add=True)

    def _scatter_wait(b):
        pltpu.make_async_copy(rows_v.at[pl.ds(b * CH, CH)],
                              acc.at[dst_v.at[b]], semc[b]).wait()

    _prefetch(0, 0)
    _prefetch(1, 1)

    def _triple(t, _):
        for j in range(NBUF):
            g = NBUF * t + j
            _wait_rows(j)
            _compute(j)
            b2 = (j + 2) % NBUF

            @pl.when(g >= 1)
            def _():
                _scatter_wait(b2)
            _prefetch(g + 2, b2)
            _scatter_start(j)
        return 0
    lax.fori_loop(0, (NCHUNK - 2) // NBUF, _triple, 0)

    for g, j in ((NCHUNK - 2, 0), (NCHUNK - 1, 1)):
        _wait_rows(j)
        _compute(j)
        _scatter_wait((j + 2) % NBUF)
        _scatter_start(j)
    _scatter_wait(1)

    plsc.subcore_barrier()
    pltpu.sync_copy(acc.at[pl.ds(sid * RPT, RPT)],
                    out_hbm.at[cid, pl.ds(sid * RPT, RPT)])

  return body


def _sc_gat(hs, sd, src, dst):
    return _sc_gat_kernel()(hs, sd, src, dst)


_GPT = (M * K) // NW


@functools.cache
def _sc_gather_kernel():
  mesh = plsc.VectorSubcoreMesh(core_axis_name="c", subcore_axis_name="s")

  @functools.partial(
    pl.kernel,
    out_type=jax.ShapeDtypeStruct((M * K, D), jnp.float32),
    mesh=mesh,
    scratch_types=[
        pltpu.VMEM((_GPT,), jnp.int32),
        pltpu.VMEM((_GPT, D), jnp.float32),
        pltpu.SemaphoreType.DMA,
    ],
    compiler_params=pltpu.CompilerParams(use_tc_tiling_on_sc=False),
  )
  def body(x_hbm, idx_hbm, out_hbm, idx_v, rows_v, sem):
    cid = lax.axis_index("c")
    sid = lax.axis_index("s")
    wid = cid * NS + sid
    base = wid * _GPT
    pltpu.sync_copy(idx_hbm.at[pl.ds(base, _GPT)], idx_v)
    pltpu.async_copy(x_hbm.at[idx_v], rows_v, sem).wait()
    pltpu.sync_copy(rows_v, out_hbm.at[pl.ds(base, _GPT)])

  return body


def _sc_gather(x, idx):
    return _sc_gather_kernel()(x, idx)


def _mm_body(ngat, with_affine, *refs):
    x_ref, w_ref, as_ref, ad_ref = refs[:4]
    nin = 4 + (2 if with_affine else 0)
    out_refs = refs[nin:]
    x = x_ref[...]
    if with_affine:
        x = x * refs[4][0:1, :] + refs[5][0:1, :]
    y = jnp.dot(x, w_ref[...], preferred_element_type=jnp.float32)
    zero8 = jnp.zeros((BLK, HEADS), jnp.float32)
    for g in range(ngat):
        h = y[:, g * D:(g + 1) * D]
        hr = h.reshape(BLK, HEADS, HID)
        ssrc = jnp.sum(hr * as_ref[...][None, g * HEADS:(g + 1) * HEADS, :], -1)
        sdst = jnp.sum(hr * ad_ref[...][None, g * HEADS:(g + 1) * HEADS, :], -1)
        out_refs[2 * g][...] = jnp.concatenate([h, ssrc, zero8], 1)
        out_refs[2 * g + 1][...] = jnp.concatenate([sdst, zero8], 1)


def _mm_proj(x, w, a_s, a_d, ngat, scale=None, shift=None):
    P = ngat * D
    with_affine = scale is not None
    outs = []
    out_specs = []
    for _ in range(ngat):
        outs.append(jax.ShapeDtypeStruct((N, ROW), jnp.float32))
        outs.append(jax.ShapeDtypeStruct((N, 16), jnp.float32))
        out_specs.append(pl.BlockSpec((BLK, ROW), lambda i: (i, 0)))
        out_specs.append(pl.BlockSpec((BLK, 16), lambda i: (i, 0)))
    in_specs = [pl.BlockSpec((BLK, D), lambda i: (i, 0)),
                pl.BlockSpec((D, P), lambda i: (0, 0)),
                pl.BlockSpec((ngat * HEADS, HID), lambda i: (0, 0)),
                pl.BlockSpec((ngat * HEADS, HID), lambda i: (0, 0))]
    ins = [x, w, a_s, a_d]
    if with_affine:
        in_specs += [pl.BlockSpec((8, D), lambda i: (0, 0)),
                     pl.BlockSpec((8, D), lambda i: (0, 0))]
        ins += [_pad8(scale), _pad8(shift)]
    return pl.pallas_call(
        functools.partial(_mm_body, ngat, with_affine),
        grid=(NBLK,),
        in_specs=in_specs,
        out_specs=out_specs,
        out_shape=tuple(outs),
    )(*ins)


def _norm_acc(a):
    s = a[0] + a[1]
    den = s[:, D:D + HEADS]
    z = s[:, :D].reshape(-1, HEADS, HID) / (den[:, :, None] + 1e-16)
    return z.reshape(-1, D)


def _post_body(nz, with_extra, *refs):
    wq_ref = refs[nz]
    bq_ref = refs[nz + 1]
    qv_ref = refs[nz + 2]
    nin = nz + 3 + (1 if with_extra else 0)
    zouts = refs[nin:nin + nz]
    wp_ref = refs[nin + nz + (1 if with_extra else 0)]
    cols = jnp.zeros((8, D), jnp.float32)
    row0 = lax.broadcasted_iota(jnp.int32, (8, D), 0) == 0
    iota = lax.broadcasted_iota(jnp.int32, (8, D), 1)
    for r in range(nz):
        z = _norm_acc(refs[r][...])
        zouts[r][...] = z
        t = jnp.tanh(jnp.dot(z, wq_ref[...],
                             preferred_element_type=jnp.float32) + bq_ref[0:1, :])
        w = jnp.sum(t * qv_ref[0:1, :], axis=1)
        cols = cols + jnp.where(row0 & (iota == r), jnp.sum(w), 0.0)
    if with_extra:
        refs[nin + nz][...] = _norm_acc(refs[nz + 3][...])
    wp_ref[...] = cols


def _post(accs, extra_acc, wq, bq, qv):
    nz = len(accs)
    with_extra = extra_acc is not None
    ins = list(accs) + [wq, bq, qv] + ([extra_acc] if with_extra else [])
    in_specs = ([pl.BlockSpec((NC, BLK, ROW), lambda i: (0, i, 0))] * nz
                + [pl.BlockSpec((D, SEMD), lambda i: (0, 0)),
                   pl.BlockSpec((8, SEMD), lambda i: (0, 0)),
                   pl.BlockSpec((8, SEMD), lambda i: (0, 0))]
                + ([pl.BlockSpec((NC, BLK, ROW), lambda i: (0, i, 0))]
                   if with_extra else []))
    outs = ([jax.ShapeDtypeStruct((N, D), jnp.float32)] * nz
            + ([jax.ShapeDtypeStruct((N, D), jnp.float32)] if with_extra else [])
            + [jax.ShapeDtypeStruct((NBLK * 8, D), jnp.float32)])
    out_specs = ([pl.BlockSpec((BLK, D), lambda i: (i, 0))] * nz
                 + ([pl.BlockSpec((BLK, D), lambda i: (i, 0))] if with_extra else [])
                 + [pl.BlockSpec((8, D), lambda i: (i, 0))])
    res = pl.pallas_call(
        functools.partial(_post_body, nz, with_extra),
        grid=(NBLK,),
        in_specs=in_specs,
        out_specs=out_specs,
        out_shape=tuple(outs),
    )(*ins)
    return res


def _combine_body(z0_ref, z1_ref, ex_ref, beta_ref, sem_ref, out_ref):
    s = beta_ref[0] * z0_ref[...] + beta_ref[1] * z1_ref[...]
    sem_ref[...] = s
    out_ref[...] = _leaky(s + ex_ref[...])


def _combine(z0, z1, extra, beta):
    return pl.pallas_call(
        _combine_body,
        grid=(NBLK,),
        in_specs=[pl.BlockSpec((BLK, D), lambda i: (i, 0)),
                  pl.BlockSpec((BLK, D), lambda i: (i, 0)),
                  pl.BlockSpec((BLK, D), lambda i: (i, 0)),
                  pl.BlockSpec(memory_space=pltpu.SMEM)],
        out_specs=[pl.BlockSpec((BLK, D), lambda i: (i, 0)),
                   pl.BlockSpec((BLK, D), lambda i: (i, 0))],
        out_shape=(jax.ShapeDtypeStruct((N, D), jnp.float32),
                   jax.ShapeDtypeStruct((N, D), jnp.float32)),
    )(z0, z1, extra, beta)


def _contrib_body(tn_ref, attn_ref, msg_ref, out_ref):
    i = pl.program_id(0)
    colid = i * BLK + lax.broadcasted_iota(jnp.int32, (M, BLK), 1)
    wt = jnp.zeros((M, BLK), jnp.float32)
    for k in range(K):
        hit = (tn_ref[:, k][:, None] == colid).astype(jnp.float32)
        wt = wt + attn_ref[:, k][:, None] * hit
    out_ref[...] = lax.dot_general(wt, msg_ref[...], (((0,), (0,)), ((), ())),
                                   preferred_element_type=jnp.float32)


def _contrib(tn, attn_pad, msg):
    return pl.pallas_call(
        _contrib_body,
        grid=(NBLK,),
        in_specs=[pl.BlockSpec((M, K), lambda i: (0, 0)),
                  pl.BlockSpec((M, D), lambda i: (0, 0)),
                  pl.BlockSpec((M, D), lambda i: (0, 0))],
        out_specs=pl.BlockSpec((BLK, D), lambda i: (i, 0)),
        out_shape=jax.ShapeDtypeStruct((N, D), jnp.float32),
    )(tn, attn_pad, msg)


def _lnbn_body(x_ref, g_ref, b_ref, y_ref, s1_ref, s2_ref):
    x = x_ref[...]
    mu = jnp.mean(x, -1, keepdims=True)
    v = jnp.mean(x * x, -1, keepdims=True) - mu * mu
    y = _leaky(g_ref[0:1, :] * (x - mu) * lax.rsqrt(v + 1e-5) + b_ref[0:1, :])
    y_ref[...] = y
    row0 = lax.broadcasted_iota(jnp.int32, (8, D), 0) == 0
    s1_ref[...] = jnp.where(row0, jnp.sum(y, axis=0, keepdims=True), 0.0)
    s2_ref[...] = jnp.where(row0, jnp.sum(y * y, axis=0, keepdims=True), 0.0)


def _lnbn_pass(x, g8, b8):
    return pl.pallas_call(
        _lnbn_body,
        grid=(NBLK,),
        in_specs=[pl.BlockSpec((BLK, D), lambda i: (i, 0)),
                  pl.BlockSpec((8, D), lambda i: (0, 0)),
                  pl.BlockSpec((8, D), lambda i: (0, 0))],
        out_specs=[pl.BlockSpec((BLK, D), lambda i: (i, 0)),
                   pl.BlockSpec((8, D), lambda i: (i, 0)),
                   pl.BlockSpec((8, D), lambda i: (i, 0))],
        out_shape=(jax.ShapeDtypeStruct((N, D), jnp.float32),
                   jax.ShapeDtypeStruct((NBLK * 8, D), jnp.float32),
                   jax.ShapeDtypeStruct((NBLK * 8, D), jnp.float32)),
    )(x, g8, b8)


def _dense_gat(h, ssrc, sdst, src, dst, n):
    oh_dst_n = (lax.broadcasted_iota(jnp.int32, (n, EMG), 0)
                == dst[None, :]).astype(jnp.float32)
    oh_src_e = (lax.broadcasted_iota(jnp.int32, (EMG, n), 1)
                == src[:, None]).astype(jnp.float32)
    oh_dst_e = (lax.broadcasted_iota(jnp.int32, (EMG, n), 1)
                == dst[:, None]).astype(jnp.float32)
    sc_src = jnp.dot(oh_src_e, ssrc, preferred_element_type=jnp.float32)
    sc_dst = jnp.dot(oh_dst_e, sdst, preferred_element_type=jnp.float32)
    e = _leaky(sc_src + sc_dst)
    big = jnp.float32(-1e30)
    m = jnp.max(jnp.where(oh_dst_n[:, :, None] > 0, e[None, :, :], big), axis=1)
    m = jnp.where(m <= big * 0.5, 0.0, m)
    ex = jnp.exp(e - jnp.dot(oh_dst_e, m, preferred_element_type=jnp.float32))
    den = jnp.dot(oh_dst_n, ex, preferred_element_type=jnp.float32)
    alpha = ex / (jnp.dot(oh_dst_e, den, preferred_element_type=jnp.float32) + 1e-16)
    hsrc = jnp.dot(oh_src_e, h, preferred_element_type=jnp.float32)
    wrow = (alpha[:, :, None] * hsrc.reshape(EMG, HEADS, HID)).reshape(EMG, D)
    return jnp.dot(oh_dst_n, wrow, preferred_element_type=jnp.float32)


def _mg_body(do_norm, *refs):
    (mgx_ref, nb_ref, mp_ref, ei_ref,
     wm_ref, bsm_ref, bdm_ref, wsem_ref, bq_ref, qv_ref,
     wpc_ref, vpc_ref, wpool_ref, wme_ref, bsme_ref, bdme_ref, wcci_ref,
     lng_ref, lnb_ref, bng_ref, bnb_ref,
     mg_out_ref, attn_out_ref, msg_ref) = refs
    x = mgx_ref[...]
    if do_norm:
        mu = jnp.mean(x, -1, keepdims=True)
        v = jnp.mean(x * x, -1, keepdims=True) - mu * mu
        x = _leaky(lng_ref[0:1, :] * (x - mu) * lax.rsqrt(v + 1e-5) + lnb_ref[0:1, :])
        mu2 = jnp.mean(x, 0, keepdims=True)
        v2 = jnp.mean(x * x, 0, keepdims=True) - mu2 * mu2
        x = bng_ref[0:1, :] * (x - mu2) * lax.rsqrt(v2 + 1e-5) + bnb_ref[0:1, :]

    mp = mp_ref[...].reshape(RREL * 2 * EMG)
    ei = ei_ref[...].reshape(2 * EMG)

    zs = []
    ws = []
    for r in range(RREL):
        w = wm_ref[...][r * D:(r + 1) * D, :]
        h = jnp.dot(x, w, preferred_element_type=jnp.float32)
        ssrc = jnp.dot(h, bsm_ref[...][r * D:(r + 1) * D, :],
                       preferred_element_type=jnp.float32)
        sdst = jnp.dot(h, bdm_ref[...][r * D:(r + 1) * D, :],
                       preferred_element_type=jnp.float32)
        src = mp[r * 2 * EMG:r * 2 * EMG + EMG]
        dst = mp[r * 2 * EMG + EMG:(r + 1) * 2 * EMG]
        z = _dense_gat(h, ssrc, sdst, src, dst, M)
        zs.append(z)
        t = jnp.tanh(jnp.dot(z, wsem_ref[...],
                             preferred_element_type=jnp.float32) + bq_ref[0:1, :])
        ws.append(jnp.mean(jnp.sum(t * qv_ref[0:1, :], axis=1)))
    w0 = ws[0]; w1 = ws[1]
    mx = jnp.maximum(w0, w1)
    e0 = jnp.exp(w0 - mx); e1 = jnp.exp(w1 - mx)
    b0 = e0 / (e0 + e1); b1 = e1 / (e0 + e1)
    mg_sem = b0 * zs[0] + b1 * zs[1]

    nb = nb_ref[...]
    t = jnp.tanh(jnp.dot(nb, wpc_ref[...], preferred_element_type=jnp.float32))
    y = jnp.sum(t * vpc_ref[0:1, :], axis=1).reshape(M, K)
    ymax = jnp.max(y, axis=1, keepdims=True)
    yex = jnp.exp(y - ymax)
    attn = yex / jnp.sum(yex, axis=1, keepdims=True)
    pooled = jnp.sum(attn[:, :, None] * nb.reshape(M, K, D), axis=1)

    he = jnp.dot(x, wme_ref[...], preferred_element_type=jnp.float32)
    ssrc = jnp.dot(he, bsme_ref[...], preferred_element_type=jnp.float32)
    sdst = jnp.dot(he, bdme_ref[...], preferred_element_type=jnp.float32)
    src = ei[0:EMG]
    dst = ei[EMG:2 * EMG]
    ge = _dense_gat(he, ssrc, sdst, src, dst, M)

    mg = _leaky(mg_sem + jnp.dot(pooled, wpool_ref[...],
                                 preferred_element_type=jnp.float32) + ge)
    mg_out_ref[...] = mg
    attn_out_ref[...] = jnp.concatenate(
        [attn, jnp.zeros((M, D - K), jnp.float32)], axis=1)
    msg_ref[...] = jnp.dot(mg, wcci_ref[...], preferred_element_type=jnp.float32)


def _mg_all(do_norm, mgx, nb, mp_i, ei_i, wm, bsm, bdm, wsem, bq, qv,
            wpc, vpc, wpool, wme, bsme, bdme, wcci, lng, lnb, bng, bnb):
    full = lambda s: pl.BlockSpec(s, lambda: tuple(0 for _ in s))
    ins = [mgx, nb, mp_i, ei_i, wm, bsm, bdm, wsem, bq, qv,
           wpc, vpc, wpool, wme, bsme, bdme, wcci, lng, lnb, bng, bnb]
    in_specs = [full(tuple(a.shape)) for a in ins]
    return pl.pallas_call(
        functools.partial(_mg_body, do_norm),
        in_specs=in_specs,
        out_specs=[full((M, D)), full((M, D)), full((M, D))],
        out_shape=(jax.ShapeDtypeStruct((M, D), jnp.float32),
                   jax.ShapeDtypeStruct((M, D), jnp.float32),
                   jax.ShapeDtypeStruct((M, D), jnp.float32)),
    )(*ins)


def _blockdiag(a):
    eye = jnp.eye(HEADS, dtype=a.dtype)
    return (eye[:, None, :] * a[:, :, None]).reshape(D, HEADS)


def _pad8(v):
    return jnp.broadcast_to(v[None, :], (8, v.shape[0]))


def _beta_from_partials(wp, n):
    t = jnp.sum(wp, axis=0)
    w = t[:2] / n
    return jax.nn.softmax(w)


def _up_layer(x, mgx, mp_srcdst, ei_srcdst, mg_mp_i, mg_ei_i, tn_flat, p,
              wcci, scale=None, shift=None, do_mg_norm=False, ln=None):
    wcat = jnp.concatenate([p['Wp'][0], p['Wp'][1], p['Wpe']], 1)
    a_s = jnp.concatenate([p['asp'][0], p['asp'][1], p['aspe']], 0)
    a_d = jnp.concatenate([p['adp'][0], p['adp'][1], p['adpe']], 0)
    hs0, sd0, hs1, sd1, hs2, sd2 = _mm_proj(x, wcat, a_s, a_d, 3,
                                            scale=scale, shift=shift)

    acc0 = _sc_gat(hs0, sd0, mp_srcdst[0][0], mp_srcdst[0][1])
    acc1 = _sc_gat(hs1, sd1, mp_srcdst[1][0], mp_srcdst[1][1])
    acc2 = _sc_gat(hs2, sd2, ei_srcdst[0], ei_srcdst[1])

    z0, z1, ze, wp = _post([acc0, acc1], acc2,
                           p['Wsem_p'], _pad8(p['bsem_p']), _pad8(p['qsem_p']))
    beta = _beta_from_partials(wp, N)
    ppi_sem, ppi_out = _combine(z0, z1, ze, beta)

    nb = _sc_gather(ppi_sem, tn_flat)
    lng, lnb, bng, bnb = ln
    mg_out, attn_pad, msg = _mg_all(
        do_mg_norm, mgx, nb, mg_mp_i, mg_ei_i,
        p['Wm'].reshape(RREL * D, D),
        jnp.concatenate([_blockdiag(p['asm'][r]) for r in range(RREL)], 0),
        jnp.concatenate([_blockdiag(p['adm'][r]) for r in range(RREL)], 0),
        p['Wsem_m'], _pad8(p['bsem_m']), _pad8(p['qsem_m']),
        p['Wpc'], _pad8(p['vpc']), p['Wpool'],
        p['Wme'], _blockdiag(p['asme']), _blockdiag(p['adme']),
        wcci,
        _pad8(lng), _pad8(lnb), _pad8(bng), _pad8(bnb))
    return ppi_out, mg_out, attn_pad, msg


def _down_layer(x, msg, attn_pad, tn, mp_srcdst, p):
    wcat = jnp.concatenate([p['Wd'][0], p['Wd'][1]], 1)
    a_s = jnp.concatenate([p['asd'][0], p['asd'][1]], 0)
    a_d = jnp.concatenate([p['add'][0], p['add'][1]], 0)
    hs0, sd0, hs1, sd1 = _mm_proj(x, wcat, a_s, a_d, 2)

    acc0 = _sc_gat(hs0, sd0, mp_srcdst[0][0], mp_srcdst[0][1])
    acc1 = _sc_gat(hs1, sd1, mp_srcdst[1][0], mp_srcdst[1][1])

    z0, z1, wp = _post([acc0, acc1], None,
                       p['Wsem_d'], _pad8(p['bsem_d']), _pad8(p['qsem_d']))
    beta = _beta_from_partials(wp, N)
    contrib = _contrib(tn, attn_pad, msg)
    _, out = _combine(z0, z1, contrib, beta)
    return out


def kernel(ppi_x, mg_x, ppi_metapaths, mg_metapaths, ppi_edge_index,
           mg_edge_index, tissue_neighbors, params):
    mp = [(ppi_metapaths[r, 0], ppi_metapaths[r, 1]) for r in range(RREL)]
    ei = (ppi_edge_index[0], ppi_edge_index[1])
    mg_mp_i = mg_metapaths.reshape(RREL * 2 * EMG // D, D)
    mg_ei_i = mg_edge_index.reshape(2 * EMG // D, D)
    tn_flat = tissue_neighbors.reshape(M * K)
    ln = (params['ln_g'], params['ln_b'], params['bn_g'], params['bn_b'])

    p1u, p1d = params['conv1_up'], params['conv1_down']
    p2u, p2d = params['conv2_up'], params['conv2_down']

    ppi1, mg1, attn1, msg1 = _up_layer(ppi_x, mg_x, mp, ei, mg_mp_i, mg_ei_i,
                                       tn_flat, p1u, p1d['Wcci'], ln=ln)
    ppi2 = _down_layer(ppi1, msg1, attn1, tissue_neighbors, mp, p1d)

    y, s1, s2 = _lnbn_pass(ppi2, _pad8(params['ln_g']), _pad8(params['ln_b']))
    mu = jnp.sum(s1, 0) / N
    var = jnp.sum(s2, 0) / N - mu * mu
    scale = params['bn_g'] * lax.rsqrt(var + 1e-5)
    shift = params['bn_b'] - mu * scale

    ppi3, mg2, attn2, msg2 = _up_layer(y, mg1, mp, ei, mg_mp_i, mg_ei_i,
                                       tn_flat, p2u, p2d['Wcci'],
                                       scale=scale, shift=shift,
                                       do_mg_norm=True, ln=ln)
    ppi4 = _down_layer(ppi3, msg2, attn2, tissue_neighbors, mp, p2d)
    return ppi4, mg2


# speedup vs baseline: 1.0343x; 1.0241x over previous
import functools

import jax
import jax.numpy as jnp
from jax import lax
from jax.experimental import pallas as pl
from jax.experimental.pallas import tpu as pltpu
from jax.experimental.pallas import tpu_sc as plsc

N = 10000; E = 160000; M = 64; K = 16; RREL = 2; D = 128; HEADS = 8; HID = 16
SEMD = 8; PC = 8; EMG = 512
ROW = D + 2 * HEADS
GROW = ROW + 16
NC, NS = 2, 16
NW = NC * NS
EPT = E // NW
CH = 200
NCHUNK = EPT // CH
RPT = N // NS
ZR = 125
BLK = 1000
NBLK = N // BLK

_LEAK = 0.2


def _leaky(x):
    return jnp.where(x > 0, x, _LEAK * x)


@functools.cache
def _sc_gat_kernel():
  mesh = plsc.VectorSubcoreMesh(core_axis_name="c", subcore_axis_name="s")

  @functools.partial(
    pl.kernel,
    out_type=jax.ShapeDtypeStruct((NC, N, ROW), jnp.float32),
    mesh=mesh,
    scratch_types=[
        pltpu.VMEM((CH,), jnp.int32),
        pltpu.VMEM((CH,), jnp.int32),
        pltpu.VMEM((CH, ROW), jnp.float32),
        pltpu.VMEM((CH, 16), jnp.float32),
        pltpu.VMEM_SHARED((N, ROW), jnp.float32),
        pltpu.SemaphoreType.DMA,
        pltpu.SemaphoreType.DMA,
    ],
    compiler_params=pltpu.CompilerParams(use_tc_tiling_on_sc=False),
  )
  def body(hs_hbm, sd_hbm, src_hbm, dst_hbm, out_hbm,
           src_v, dst_v, hs_rows, sd_rows, acc,
           sem1, sem2):
    cid = lax.axis_index("c")
    sid = lax.axis_index("s")
    wid = cid * NS + sid

    def _zrow(i, _):
        for j in range(ROW // 16):
            hs_rows[i, pl.ds(16 * j, 16)] = jnp.zeros((16,), jnp.float32)
        return 0
    lax.fori_loop(0, CH, _zrow, 0)
    for b in range(RPT // CH):
        pltpu.sync_copy(hs_rows, acc.at[pl.ds(sid * RPT + b * CH, CH)])
    rem = RPT % CH
    if rem:
        pltpu.sync_copy(hs_rows.at[pl.ds(0, rem)],
                        acc.at[pl.ds(sid * RPT + (RPT // CH) * CH, rem)])
    plsc.subcore_barrier()

    lanes = lax.iota(jnp.int32, 16)
    head_mask = lanes < HEADS

    def _chunk(g, _):
        base = wid * EPT + g * CH
        pltpu.sync_copy(src_hbm.at[pl.ds(base, CH)], src_v)
        pltpu.sync_copy(dst_hbm.at[pl.ds(base, CH)], dst_v)
        cp1 = pltpu.async_copy(hs_hbm.at[src_v], hs_rows, sem1)
        cp2 = pltpu.async_copy(sd_hbm.at[dst_v], sd_rows, sem2)
        cp1.wait()
        cp2.wait()

        def _edge(i, _):
            ssrc = hs_rows[i, pl.ds(D, 16)]
            sdst = sd_rows[i]
            e = ssrc + sdst
            e = jnp.where(e > 0, e, _LEAK * e)
            ex = jnp.where(head_mask, jnp.exp(e), 0.0)
            hs_rows[i, pl.ds(D, 16)] = ex
            for j in range(HEADS):
                a = jnp.full((16,), ex[j], jnp.float32)
                hs_rows[i, pl.ds(16 * j, 16)] = hs_rows[i, pl.ds(16 * j, 16)] * a
            return 0
        lax.fori_loop(0, CH, _edge, 0, unroll=2)
        pltpu.sync_copy(hs_rows, acc.at[dst_v], add=True)
        return 0
    lax.fori_loop(0, NCHUNK, _chunk, 0)

    plsc.subcore_barrier()
    pltpu.sync_copy(acc.at[pl.ds(sid * RPT, RPT)],
                    out_hbm.at[cid, pl.ds(sid * RPT, RPT)])

  return body


def _sc_gat(hs, sd, src, dst):
    return _sc_gat_kernel()(hs, sd, src, dst)


_GPT = (M * K) // NW


@functools.cache
def _sc_gather_kernel():
  mesh = plsc.VectorSubcoreMesh(core_axis_name="c", subcore_axis_name="s")

  @functools.partial(
    pl.kernel,
    out_type=jax.ShapeDtypeStruct((M * K, D), jnp.float32),
    mesh=mesh,
    scratch_types=[
        pltpu.VMEM((_GPT,), jnp.int32),
        pltpu.VMEM((_GPT, D), jnp.float32),
        pltpu.SemaphoreType.DMA,
    ],
    compiler_params=pltpu.CompilerParams(use_tc_tiling_on_sc=False),
  )
  def body(x_hbm, idx_hbm, out_hbm, idx_v, rows_v, sem):
    cid = lax.axis_index("c")
    sid = lax.axis_index("s")
    wid = cid * NS + sid
    base = wid * _GPT
    pltpu.sync_copy(idx_hbm.at[pl.ds(base, _GPT)], idx_v)
    pltpu.async_copy(x_hbm.at[idx_v], rows_v, sem).wait()
    pltpu.sync_copy(rows_v, out_hbm.at[pl.ds(base, _GPT)])

  return body


def _sc_gather(x, idx):
    return _sc_gather_kernel()(x, idx)


def _mm_body(ngat, with_affine, *refs):
    x_ref, w_ref, as_ref, ad_ref = refs[:4]
    nin = 4 + (2 if with_affine else 0)
    out_refs = refs[nin:]
    x = x_ref[...]
    if with_affine:
        x = x * refs[4][0:1, :] + refs[5][0:1, :]
    y = jnp.dot(x, w_ref[...], preferred_element_type=jnp.float32)
    zero8 = jnp.zeros((BLK, HEADS), jnp.float32)
    for g in range(ngat):
        h = y[:, g * D:(g + 1) * D]
        hr = h.reshape(BLK, HEADS, HID)
        ssrc = jnp.sum(hr * as_ref[...][None, g * HEADS:(g + 1) * HEADS, :], -1)
        sdst = jnp.sum(hr * ad_ref[...][None, g * HEADS:(g + 1) * HEADS, :], -1)
        out_refs[2 * g][...] = jnp.concatenate([h, ssrc, zero8], 1)
        out_refs[2 * g + 1][...] = jnp.concatenate([sdst, zero8], 1)


def _mm_proj(x, w, a_s, a_d, ngat, scale=None, shift=None):
    P = ngat * D
    with_affine = scale is not None
    outs = []
    out_specs = []
    for _ in range(ngat):
        outs.append(jax.ShapeDtypeStruct((N, ROW), jnp.float32))
        outs.append(jax.ShapeDtypeStruct((N, 16), jnp.float32))
        out_specs.append(pl.BlockSpec((BLK, ROW), lambda i: (i, 0)))
        out_specs.append(pl.BlockSpec((BLK, 16), lambda i: (i, 0)))
    in_specs = [pl.BlockSpec((BLK, D), lambda i: (i, 0)),
                pl.BlockSpec((D, P), lambda i: (0, 0)),
                pl.BlockSpec((ngat * HEADS, HID), lambda i: (0, 0)),
                pl.BlockSpec((ngat * HEADS, HID), lambda i: (0, 0))]
    ins = [x, w, a_s, a_d]
    if with_affine:
        in_specs += [pl.BlockSpec((8, D), lambda i: (0, 0)),
                     pl.BlockSpec((8, D), lambda i: (0, 0))]
        ins += [_pad8(scale), _pad8(shift)]
    return pl.pallas_call(
        functools.partial(_mm_body, ngat, with_affine),
        grid=(NBLK,),
        in_specs=in_specs,
        out_specs=out_specs,
        out_shape=tuple(outs),
    )(*ins)


def _norm_acc(a):
    s = a[0] + a[1]
    den = s[:, D:D + HEADS]
    z = s[:, :D].reshape(-1, HEADS, HID) / (den[:, :, None] + 1e-16)
    return z.reshape(-1, D)


def _post_body(nz, with_extra, *refs):
    wq_ref = refs[nz]
    bq_ref = refs[nz + 1]
    qv_ref = refs[nz + 2]
    nin = nz + 3 + (1 if with_extra else 0)
    zouts = refs[nin:nin + nz]
    wp_ref = refs[nin + nz + (1 if with_extra else 0)]
    cols = jnp.zeros((8, D), jnp.float32)
    row0 = lax.broadcasted_iota(jnp.int32, (8, D), 0) == 0
    iota = lax.broadcasted_iota(jnp.int32, (8, D), 1)
    for r in range(nz):
        z = _norm_acc(refs[r][...])
        zouts[r][...] = z
        t = jnp.tanh(jnp.dot(z, wq_ref[...],
                             preferred_element_type=jnp.float32) + bq_ref[0:1, :])
        w = jnp.sum(t * qv_ref[0:1, :], axis=1)
        cols = cols + jnp.where(row0 & (iota == r), jnp.sum(w), 0.0)
    if with_extra:
        refs[nin + nz][...] = _norm_acc(refs[nz + 3][...])
    wp_ref[...] = cols


def _post(accs, extra_acc, wq, bq, qv):
    nz = len(accs)
    with_extra = extra_acc is not None
    ins = list(accs) + [wq, bq, qv] + ([extra_acc] if with_extra else [])
    in_specs = ([pl.BlockSpec((NC, BLK, ROW), lambda i: (0, i, 0))] * nz
                + [pl.BlockSpec((D, SEMD), lambda i: (0, 0)),
                   pl.BlockSpec((8, SEMD), lambda i: (0, 0)),
                   pl.BlockSpec((8, SEMD), lambda i: (0, 0))]
                + ([pl.BlockSpec((NC, BLK, ROW), lambda i: (0, i, 0))]
                   if with_extra else []))
    outs = ([jax.ShapeDtypeStruct((N, D), jnp.float32)] * nz
            + ([jax.ShapeDtypeStruct((N, D), jnp.float32)] if with_extra else [])
            + [jax.ShapeDtypeStruct((NBLK * 8, D), jnp.float32)])
    out_specs = ([pl.BlockSpec((BLK, D), lambda i: (i, 0))] * nz
                 + ([pl.BlockSpec((BLK, D), lambda i: (i, 0))] if with_extra else [])
                 + [pl.BlockSpec((8, D), lambda i: (i, 0))])
    res = pl.pallas_call(
        functools.partial(_post_body, nz, with_extra),
        grid=(NBLK,),
        in_specs=in_specs,
        out_specs=out_specs,
        out_shape=tuple(outs),
    )(*ins)
    return res


def _combine_body(z0_ref, z1_ref, ex_ref, beta_ref, sem_ref, out_ref):
    s = beta_ref[0] * z0_ref[...] + beta_ref[1] * z1_ref[...]
    sem_ref[...] = s
    out_ref[...] = _leaky(s + ex_ref[...])


def _combine(z0, z1, extra, beta):
    return pl.pallas_call(
        _combine_body,
        grid=(NBLK,),
        in_specs=[pl.BlockSpec((BLK, D), lambda i: (i, 0)),
                  pl.BlockSpec((BLK, D), lambda i: (i, 0)),
                  pl.BlockSpec((BLK, D), lambda i: (i, 0)),
                  pl.BlockSpec(memory_space=pltpu.SMEM)],
        out_specs=[pl.BlockSpec((BLK, D), lambda i: (i, 0)),
                   pl.BlockSpec((BLK, D), lambda i: (i, 0))],
        out_shape=(jax.ShapeDtypeStruct((N, D), jnp.float32),
                   jax.ShapeDtypeStruct((N, D), jnp.float32)),
    )(z0, z1, extra, beta)


def _contrib_body(tn_ref, attn_ref, msg_ref, out_ref):
    i = pl.program_id(0)
    colid = i * BLK + lax.broadcasted_iota(jnp.int32, (M, BLK), 1)
    wt = jnp.zeros((M, BLK), jnp.float32)
    for k in range(K):
        hit = (tn_ref[:, k][:, None] == colid).astype(jnp.float32)
        wt = wt + attn_ref[:, k][:, None] * hit
    out_ref[...] = lax.dot_general(wt, msg_ref[...], (((0,), (0,)), ((), ())),
                                   preferred_element_type=jnp.float32)


def _contrib(tn, attn_pad, msg):
    return pl.pallas_call(
        _contrib_body,
        grid=(NBLK,),
        in_specs=[pl.BlockSpec((M, K), lambda i: (0, 0)),
                  pl.BlockSpec((M, D), lambda i: (0, 0)),
                  pl.BlockSpec((M, D), lambda i: (0, 0))],
        out_specs=pl.BlockSpec((BLK, D), lambda i: (i, 0)),
        out_shape=jax.ShapeDtypeStruct((N, D), jnp.float32),
    )(tn, attn_pad, msg)


def _lnbn_body(x_ref, g_ref, b_ref, y_ref, s1_ref, s2_ref):
    x = x_ref[...]
    mu = jnp.mean(x, -1, keepdims=True)
    v = jnp.mean(x * x, -1, keepdims=True) - mu * mu
    y = _leaky(g_ref[0:1, :] * (x - mu) * lax.rsqrt(v + 1e-5) + b_ref[0:1, :])
    y_ref[...] = y
    row0 = lax.broadcasted_iota(jnp.int32, (8, D), 0) == 0
    s1_ref[...] = jnp.where(row0, jnp.sum(y, axis=0, keepdims=True), 0.0)
    s2_ref[...] = jnp.where(row0, jnp.sum(y * y, axis=0, keepdims=True), 0.0)


def _lnbn_pass(x, g8, b8):
    return pl.pallas_call(
        _lnbn_body,
        grid=(NBLK,),
        in_specs=[pl.BlockSpec((BLK, D), lambda i: (i, 0)),
                  pl.BlockSpec((8, D), lambda i: (0, 0)),
                  pl.BlockSpec((8, D), lambda i: (0, 0))],
        out_specs=[pl.BlockSpec((BLK, D), lambda i: (i, 0)),
                   pl.BlockSpec((8, D), lambda i: (i, 0)),
                   pl.BlockSpec((8, D), lambda i: (i, 0))],
        out_shape=(jax.ShapeDtypeStruct((N, D), jnp.float32),
                   jax.ShapeDtypeStruct((NBLK * 8, D), jnp.float32),
                   jax.ShapeDtypeStruct((NBLK * 8, D), jnp.float32)),
    )(x, g8, b8)


def _dense_gat(h, ssrc, sdst, src, dst, n):
    oh_dst_n = (lax.broadcasted_iota(jnp.int32, (n, EMG), 0)
                == dst[None, :]).astype(jnp.float32)
    oh_src_e = (lax.broadcasted_iota(jnp.int32, (EMG, n), 1)
                == src[:, None]).astype(jnp.float32)
    oh_dst_e = (lax.broadcasted_iota(jnp.int32, (EMG, n), 1)
                == dst[:, None]).astype(jnp.float32)
    sc_src = jnp.dot(oh_src_e, ssrc, preferred_element_type=jnp.float32)
    sc_dst = jnp.dot(oh_dst_e, sdst, preferred_element_type=jnp.float32)
    e = _leaky(sc_src + sc_dst)
    big = jnp.float32(-1e30)
    m = jnp.max(jnp.where(oh_dst_n[:, :, None] > 0, e[None, :, :], big), axis=1)
    m = jnp.where(m <= big * 0.5, 0.0, m)
    ex = jnp.exp(e - jnp.dot(oh_dst_e, m, preferred_element_type=jnp.float32))
    den = jnp.dot(oh_dst_n, ex, preferred_element_type=jnp.float32)
    alpha = ex / (jnp.dot(oh_dst_e, den, preferred_element_type=jnp.float32) + 1e-16)
    hsrc = jnp.dot(oh_src_e, h, preferred_element_type=jnp.float32)
    wrow = (alpha[:, :, None] * hsrc.reshape(EMG, HEADS, HID)).reshape(EMG, D)
    return jnp.dot(oh_dst_n, wrow, preferred_element_type=jnp.float32)


def _mg_body(do_norm, *refs):
    (mgx_ref, nb_ref, mp_ref, ei_ref,
     wm_ref, bsm_ref, bdm_ref, wsem_ref, bq_ref, qv_ref,
     wpc_ref, vpc_ref, wpool_ref, wme_ref, bsme_ref, bdme_ref, wcci_ref,
     lng_ref, lnb_ref, bng_ref, bnb_ref,
     mg_out_ref, attn_out_ref, msg_ref) = refs
    x = mgx_ref[...]
    if do_norm:
        mu = jnp.mean(x, -1, keepdims=True)
        v = jnp.mean(x * x, -1, keepdims=True) - mu * mu
        x = _leaky(lng_ref[0:1, :] * (x - mu) * lax.rsqrt(v + 1e-5) + lnb_ref[0:1, :])
        mu2 = jnp.mean(x, 0, keepdims=True)
        v2 = jnp.mean(x * x, 0, keepdims=True) - mu2 * mu2
        x = bng_ref[0:1, :] * (x - mu2) * lax.rsqrt(v2 + 1e-5) + bnb_ref[0:1, :]

    mp = mp_ref[...].reshape(RREL * 2 * EMG)
    ei = ei_ref[...].reshape(2 * EMG)

    zs = []
    ws = []
    for r in range(RREL):
        w = wm_ref[...][r * D:(r + 1) * D, :]
        h = jnp.dot(x, w, preferred_element_type=jnp.float32)
        ssrc = jnp.dot(h, bsm_ref[...][r * D:(r + 1) * D, :],
                       preferred_element_type=jnp.float32)
        sdst = jnp.dot(h, bdm_ref[...][r * D:(r + 1) * D, :],
                       preferred_element_type=jnp.float32)
        src = mp[r * 2 * EMG:r * 2 * EMG + EMG]
        dst = mp[r * 2 * EMG + EMG:(r + 1) * 2 * EMG]
        z = _dense_gat(h, ssrc, sdst, src, dst, M)
        zs.append(z)
        t = jnp.tanh(jnp.dot(z, wsem_ref[...],
                             preferred_element_type=jnp.float32) + bq_ref[0:1, :])
        ws.append(jnp.mean(jnp.sum(t * qv_ref[0:1, :], axis=1)))
    w0 = ws[0]; w1 = ws[1]
    mx = jnp.maximum(w0, w1)
    e0 = jnp.exp(w0 - mx); e1 = jnp.exp(w1 - mx)
    b0 = e0 / (e0 + e1); b1 = e1 / (e0 + e1)
    mg_sem = b0 * zs[0] + b1 * zs[1]

    nb = nb_ref[...]
    t = jnp.tanh(jnp.dot(nb, wpc_ref[...], preferred_element_type=jnp.float32))
    y = jnp.sum(t * vpc_ref[0:1, :], axis=1).reshape(M, K)
    ymax = jnp.max(y, axis=1, keepdims=True)
    yex = jnp.exp(y - ymax)
    attn = yex / jnp.sum(yex, axis=1, keepdims=True)
    pooled = jnp.sum(attn[:, :, None] * nb.reshape(M, K, D), axis=1)

    he = jnp.dot(x, wme_ref[...], preferred_element_type=jnp.float32)
    ssrc = jnp.dot(he, bsme_ref[...], preferred_element_type=jnp.float32)
    sdst = jnp.dot(he, bdme_ref[...], preferred_element_type=jnp.float32)
    src = ei[0:EMG]
    dst = ei[EMG:2 * EMG]
    ge = _dense_gat(he, ssrc, sdst, src, dst, M)

    mg = _leaky(mg_sem + jnp.dot(pooled, wpool_ref[...],
                                 preferred_element_type=jnp.float32) + ge)
    mg_out_ref[...] = mg
    attn_out_ref[...] = jnp.concatenate(
        [attn, jnp.zeros((M, D - K), jnp.float32)], axis=1)
    msg_ref[...] = jnp.dot(mg, wcci_ref[...], preferred_element_type=jnp.float32)


def _mg_all(do_norm, mgx, nb, mp_i, ei_i, wm, bsm, bdm, wsem, bq, qv,
            wpc, vpc, wpool, wme, bsme, bdme, wcci, lng, lnb, bng, bnb):
    full = lambda s: pl.BlockSpec(s, lambda: tuple(0 for _ in s))
    ins = [mgx, nb, mp_i, ei_i, wm, bsm, bdm, wsem, bq, qv,
           wpc, vpc, wpool, wme, bsme, bdme, wcci, lng, lnb, bng, bnb]
    in_specs = [full(tuple(a.shape)) for a in ins]
    return pl.pallas_call(
        functools.partial(_mg_body, do_norm),
        in_specs=in_specs,
        out_specs=[full((M, D)), full((M, D)), full((M, D))],
        out_shape=(jax.ShapeDtypeStruct((M, D), jnp.float32),
                   jax.ShapeDtypeStruct((M, D), jnp.float32),
                   jax.ShapeDtypeStruct((M, D), jnp.float32)),
    )(*ins)


def _blockdiag(a):
    eye = jnp.eye(HEADS, dtype=a.dtype)
    return (eye[:, None, :] * a[:, :, None]).reshape(D, HEADS)


def _pad8(v):
    return jnp.broadcast_to(v[None, :], (8, v.shape[0]))


def _beta_from_partials(wp, n):
    t = jnp.sum(wp, axis=0)
    w = t[:2] / n
    return jax.nn.softmax(w)


def _up_layer(x, mgx, mp_srcdst, ei_srcdst, mg_mp_i, mg_ei_i, tn_flat, p,
              wcci, scale=None, shift=None, do_mg_norm=False, ln=None):
    wcat = jnp.concatenate([p['Wp'][0], p['Wp'][1], p['Wpe']], 1)
    a_s = jnp.concatenate([p['asp'][0], p['asp'][1], p['aspe']], 0)
    a_d = jnp.concatenate([p['adp'][0], p['adp'][1], p['adpe']], 0)
    hs0, sd0, hs1, sd1, hs2, sd2 = _mm_proj(x, wcat, a_s, a_d, 3,
                                            scale=scale, shift=shift)

    acc0 = _sc_gat(hs0, sd0, mp_srcdst[0][0], mp_srcdst[0][1])
    acc1 = _sc_gat(hs1, sd1, mp_srcdst[1][0], mp_srcdst[1][1])
    acc2 = _sc_gat(hs2, sd2, ei_srcdst[0], ei_srcdst[1])

    z0, z1, ze, wp = _post([acc0, acc1], acc2,
                           p['Wsem_p'], _pad8(p['bsem_p']), _pad8(p['qsem_p']))
    beta = _beta_from_partials(wp, N)
    ppi_sem, ppi_out = _combine(z0, z1, ze, beta)

    nb = _sc_gather(ppi_sem, tn_flat)
    lng, lnb, bng, bnb = ln
    mg_out, attn_pad, msg = _mg_all(
        do_mg_norm, mgx, nb, mg_mp_i, mg_ei_i,
        p['Wm'].reshape(RREL * D, D),
        jnp.concatenate([_blockdiag(p['asm'][r]) for r in range(RREL)], 0),
        jnp.concatenate([_blockdiag(p['adm'][r]) for r in range(RREL)], 0),
        p['Wsem_m'], _pad8(p['bsem_m']), _pad8(p['qsem_m']),
        p['Wpc'], _pad8(p['vpc']), p['Wpool'],
        p['Wme'], _blockdiag(p['asme']), _blockdiag(p['adme']),
        wcci,
        _pad8(lng), _pad8(lnb), _pad8(bng), _pad8(bnb))
    return ppi_out, mg_out, attn_pad, msg


def _down_layer(x, msg, attn_pad, tn, mp_srcdst, p):
    wcat = jnp.concatenate([p['Wd'][0], p['Wd'][1]], 1)
    a_s = jnp.concatenate([p['asd'][0], p['asd'][1]], 0)
    a_d = jnp.concatenate([p['add'][0], p['add'][1]], 0)
    hs0, sd0, hs1, sd1 = _mm_proj(x, wcat, a_s, a_d, 2)

    acc0 = _sc_gat(hs0, sd0, mp_srcdst[0][0], mp_srcdst[0][1])
    acc1 = _sc_gat(hs1, sd1, mp_srcdst[1][0], mp_srcdst[1][1])

    z0, z1, wp = _post([acc0, acc1], None,
                       p['Wsem_d'], _pad8(p['bsem_d']), _pad8(p['qsem_d']))
    beta = _beta_from_partials(wp, N)
    contrib = _contrib(tn, attn_pad, msg)
    _, out = _combine(z0, z1, contrib, beta)
    return out


def kernel(ppi_x, mg_x, ppi_metapaths, mg_metapaths, ppi_edge_index,
           mg_edge_index, tissue_neighbors, params):
    mp = [(ppi_metapaths[r, 0], ppi_metapaths[r, 1]) for r in range(RREL)]
    ei = (ppi_edge_index[0], ppi_edge_index[1])
    mg_mp_i = mg_metapaths.reshape(RREL * 2 * EMG // D, D)
    mg_ei_i = mg_edge_index.reshape(2 * EMG // D, D)
    tn_flat = tissue_neighbors.reshape(M * K)
    ln = (params['ln_g'], params['ln_b'], params['bn_g'], params['bn_b'])

    p1u, p1d = params['conv1_up'], params['conv1_down']
    p2u, p2d = params['conv2_up'], params['conv2_down']

    ppi1, mg1, attn1, msg1 = _up_layer(ppi_x, mg_x, mp, ei, mg_mp_i, mg_ei_i,
                                       tn_flat, p1u, p1d['Wcci'], ln=ln)
    ppi2 = _down_layer(ppi1, msg1, attn1, tissue_neighbors, mp, p1d)

    y, s1, s2 = _lnbn_pass(ppi2, _pad8(params['ln_g']), _pad8(params['ln_b']))
    mu = jnp.sum(s1, 0) / N
    var = jnp.sum(s2, 0) / N - mu * mu
    scale = params['bn_g'] * lax.rsqrt(var + 1e-5)
    shift = params['bn_b'] - mu * scale

    ppi3, mg2, attn2, msg2 = _up_layer(y, mg1, mp, ei, mg_mp_i, mg_ei_i,
                                       tn_flat, p2u, p2d['Wcci'],
                                       scale=scale, shift=shift,
                                       do_mg_norm=True, ln=ln)
    ppi4 = _down_layer(ppi3, msg2, attn2, tissue_neighbors, mp, p2d)
    return ppi4, mg2


# speedup vs baseline: 1.0393x; 1.0049x over previous
import functools

import jax
import jax.numpy as jnp
from jax import lax
from jax.experimental import pallas as pl
from jax.experimental.pallas import tpu as pltpu
from jax.experimental.pallas import tpu_sc as plsc

N = 10000; E = 160000; M = 64; K = 16; RREL = 2; D = 128; HEADS = 8; HID = 16
SEMD = 8; PC = 8; EMG = 512
ROW = D + 2 * HEADS
GROW = ROW + 16
NC, NS = 2, 16
NW = NC * NS
EPT = E // NW
CH = 200
NCHUNK = EPT // CH
RPT = N // NS
ZR = 125
BLK = 1000
NBLK = N // BLK

_LEAK = 0.2


def _leaky(x):
    return jnp.where(x > 0, x, _LEAK * x)


@functools.cache
def _sc_gat_kernel():
  mesh = plsc.VectorSubcoreMesh(core_axis_name="c", subcore_axis_name="s")

  @functools.partial(
    pl.kernel,
    out_type=jax.ShapeDtypeStruct((NC, N, ROW), jnp.float32),
    mesh=mesh,
    scratch_types=[
        pltpu.VMEM((CH,), jnp.int32),
        pltpu.VMEM((CH,), jnp.int32),
        pltpu.VMEM((CH, ROW), jnp.float32),
        pltpu.VMEM((CH, 16), jnp.float32),
        pltpu.VMEM_SHARED((N, ROW), jnp.float32),
        pltpu.SemaphoreType.DMA,
        pltpu.SemaphoreType.DMA,
    ],
    compiler_params=pltpu.CompilerParams(use_tc_tiling_on_sc=False),
  )
  def body(hs_hbm, sd_hbm, src_hbm, dst_hbm, out_hbm,
           src_v, dst_v, hs_rows, sd_rows, acc,
           sem1, sem2):
    cid = lax.axis_index("c")
    sid = lax.axis_index("s")
    wid = cid * NS + sid

    def _zrow(i, _):
        for j in range(ROW // 16):
            hs_rows[i, pl.ds(16 * j, 16)] = jnp.zeros((16,), jnp.float32)
        return 0
    lax.fori_loop(0, CH, _zrow, 0)
    for b in range(RPT // CH):
        pltpu.sync_copy(hs_rows, acc.at[pl.ds(sid * RPT + b * CH, CH)])
    rem = RPT % CH
    if rem:
        pltpu.sync_copy(hs_rows.at[pl.ds(0, rem)],
                        acc.at[pl.ds(sid * RPT + (RPT // CH) * CH, rem)])
    plsc.subcore_barrier()

    lanes = lax.iota(jnp.int32, 16)
    head_mask = lanes < HEADS

    def _chunk(g, _):
        base = wid * EPT + g * CH
        pltpu.sync_copy(src_hbm.at[pl.ds(base, CH)], src_v)
        pltpu.sync_copy(dst_hbm.at[pl.ds(base, CH)], dst_v)
        cp1 = pltpu.async_copy(hs_hbm.at[src_v], hs_rows, sem1)
        cp2 = pltpu.async_copy(sd_hbm.at[dst_v], sd_rows, sem2)
        cp1.wait()
        cp2.wait()

        def _edge(i, _):
            ssrc = hs_rows[i, pl.ds(D, 16)]
            sdst = sd_rows[i]
            e = ssrc + sdst
            e = jnp.where(e > 0, e, _LEAK * e)
            ex = jnp.where(head_mask, jnp.exp(e), 0.0)
            hs_rows[i, pl.ds(D, 16)] = ex
            for j in range(HEADS):
                a = jnp.full((16,), ex[j], jnp.float32)
                hs_rows[i, pl.ds(16 * j, 16)] = hs_rows[i, pl.ds(16 * j, 16)] * a
            return 0
        lax.fori_loop(0, CH, _edge, 0)
        pltpu.sync_copy(hs_rows, acc.at[dst_v], add=True)
        return 0
    lax.fori_loop(0, NCHUNK, _chunk, 0)

    plsc.subcore_barrier()
    pltpu.sync_copy(acc.at[pl.ds(sid * RPT, RPT)],
                    out_hbm.at[cid, pl.ds(sid * RPT, RPT)])

  return body


def _sc_gat(hs, sd, src, dst):
    return _sc_gat_kernel()(hs, sd, src, dst)


_GPT = (M * K) // NW


@functools.cache
def _sc_gather_kernel():
  mesh = plsc.VectorSubcoreMesh(core_axis_name="c", subcore_axis_name="s")

  @functools.partial(
    pl.kernel,
    out_type=jax.ShapeDtypeStruct((M * K, D), jnp.float32),
    mesh=mesh,
    scratch_types=[
        pltpu.VMEM((_GPT,), jnp.int32),
        pltpu.VMEM((_GPT, D), jnp.float32),
        pltpu.SemaphoreType.DMA,
    ],
    compiler_params=pltpu.CompilerParams(use_tc_tiling_on_sc=False),
  )
  def body(x_hbm, idx_hbm, out_hbm, idx_v, rows_v, sem):
    cid = lax.axis_index("c")
    sid = lax.axis_index("s")
    wid = cid * NS + sid
    base = wid * _GPT
    pltpu.sync_copy(idx_hbm.at[pl.ds(base, _GPT)], idx_v)
    pltpu.async_copy(x_hbm.at[idx_v], rows_v, sem).wait()
    pltpu.sync_copy(rows_v, out_hbm.at[pl.ds(base, _GPT)])

  return body


def _sc_gather(x, idx):
    return _sc_gather_kernel()(x, idx)


def _mm_body(ngat, with_affine, *refs):
    x_ref, w_ref, as_ref, ad_ref = refs[:4]
    nin = 4 + (2 if with_affine else 0)
    out_refs = refs[nin:]
    x = x_ref[...]
    if with_affine:
        x = x * refs[4][0:1, :] + refs[5][0:1, :]
    y = jnp.dot(x, w_ref[...], preferred_element_type=jnp.float32)
    zero8 = jnp.zeros((BLK, HEADS), jnp.float32)
    for g in range(ngat):
        h = y[:, g * D:(g + 1) * D]
        hr = h.reshape(BLK, HEADS, HID)
        ssrc = jnp.sum(hr * as_ref[...][None, g * HEADS:(g + 1) * HEADS, :], -1)
        sdst = jnp.sum(hr * ad_ref[...][None, g * HEADS:(g + 1) * HEADS, :], -1)
        out_refs[2 * g][...] = jnp.concatenate([h, ssrc, zero8], 1)
        out_refs[2 * g + 1][...] = jnp.concatenate([sdst, zero8], 1)


def _mm_proj(x, w, a_s, a_d, ngat, scale=None, shift=None):
    P = ngat * D
    with_affine = scale is not None
    outs = []
    out_specs = []
    for _ in range(ngat):
        outs.append(jax.ShapeDtypeStruct((N, ROW), jnp.float32))
        outs.append(jax.ShapeDtypeStruct((N, 16), jnp.float32))
        out_specs.append(pl.BlockSpec((BLK, ROW), lambda i: (i, 0)))
        out_specs.append(pl.BlockSpec((BLK, 16), lambda i: (i, 0)))
    in_specs = [pl.BlockSpec((BLK, D), lambda i: (i, 0)),
                pl.BlockSpec((D, P), lambda i: (0, 0)),
                pl.BlockSpec((ngat * HEADS, HID), lambda i: (0, 0)),
                pl.BlockSpec((ngat * HEADS, HID), lambda i: (0, 0))]
    ins = [x, w, a_s, a_d]
    if with_affine:
        in_specs += [pl.BlockSpec((8, D), lambda i: (0, 0)),
                     pl.BlockSpec((8, D), lambda i: (0, 0))]
        ins += [_pad8(scale), _pad8(shift)]
    return pl.pallas_call(
        functools.partial(_mm_body, ngat, with_affine),
        grid=(NBLK,),
        in_specs=in_specs,
        out_specs=out_specs,
        out_shape=tuple(outs),
    )(*ins)


def _norm_acc(a):
    s = a[0] + a[1]
    den = s[:, D:D + HEADS]
    z = s[:, :D].reshape(-1, HEADS, HID) / (den[:, :, None] + 1e-16)
    return z.reshape(-1, D)


def _post_body(nz, with_extra, *refs):
    wq_ref = refs[nz]
    bq_ref = refs[nz + 1]
    qv_ref = refs[nz + 2]
    nin = nz + 3 + (1 if with_extra else 0)
    zouts = refs[nin:nin + nz]
    wp_ref = refs[nin + nz + (1 if with_extra else 0)]
    cols = jnp.zeros((8, D), jnp.float32)
    row0 = lax.broadcasted_iota(jnp.int32, (8, D), 0) == 0
    iota = lax.broadcasted_iota(jnp.int32, (8, D), 1)
    for r in range(nz):
        z = _norm_acc(refs[r][...])
        zouts[r][...] = z
        t = jnp.tanh(jnp.dot(z, wq_ref[...],
                             preferred_element_type=jnp.float32) + bq_ref[0:1, :])
        w = jnp.sum(t * qv_ref[0:1, :], axis=1)
        cols = cols + jnp.where(row0 & (iota == r), jnp.sum(w), 0.0)
    if with_extra:
        refs[nin + nz][...] = _norm_acc(refs[nz + 3][...])
    wp_ref[...] = cols


def _post(accs, extra_acc, wq, bq, qv):
    nz = len(accs)
    with_extra = extra_acc is not None
    ins = list(accs) + [wq, bq, qv] + ([extra_acc] if with_extra else [])
    in_specs = ([pl.BlockSpec((NC, BLK, ROW), lambda i: (0, i, 0))] * nz
                + [pl.BlockSpec((D, SEMD), lambda i: (0, 0)),
                   pl.BlockSpec((8, SEMD), lambda i: (0, 0)),
                   pl.BlockSpec((8, SEMD), lambda i: (0, 0))]
                + ([pl.BlockSpec((NC, BLK, ROW), lambda i: (0, i, 0))]
                   if with_extra else []))
    outs = ([jax.ShapeDtypeStruct((N, D), jnp.float32)] * nz
            + ([jax.ShapeDtypeStruct((N, D), jnp.float32)] if with_extra else [])
            + [jax.ShapeDtypeStruct((NBLK * 8, D), jnp.float32)])
    out_specs = ([pl.BlockSpec((BLK, D), lambda i: (i, 0))] * nz
                 + ([pl.BlockSpec((BLK, D), lambda i: (i, 0))] if with_extra else [])
                 + [pl.BlockSpec((8, D), lambda i: (i, 0))])
    res = pl.pallas_call(
        functools.partial(_post_body, nz, with_extra),
        grid=(NBLK,),
        in_specs=in_specs,
        out_specs=out_specs,
        out_shape=tuple(outs),
    )(*ins)
    return res


def _combine_body(z0_ref, z1_ref, ex_ref, beta_ref, sem_ref, out_ref):
    s = beta_ref[0] * z0_ref[...] + beta_ref[1] * z1_ref[...]
    sem_ref[...] = s
    out_ref[...] = _leaky(s + ex_ref[...])


def _combine(z0, z1, extra, beta):
    return pl.pallas_call(
        _combine_body,
        grid=(NBLK,),
        in_specs=[pl.BlockSpec((BLK, D), lambda i: (i, 0)),
                  pl.BlockSpec((BLK, D), lambda i: (i, 0)),
                  pl.BlockSpec((BLK, D), lambda i: (i, 0)),
                  pl.BlockSpec(memory_space=pltpu.SMEM)],
        out_specs=[pl.BlockSpec((BLK, D), lambda i: (i, 0)),
                   pl.BlockSpec((BLK, D), lambda i: (i, 0))],
        out_shape=(jax.ShapeDtypeStruct((N, D), jnp.float32),
                   jax.ShapeDtypeStruct((N, D), jnp.float32)),
    )(z0, z1, extra, beta)


def _contrib_body(tn_ref, attn_ref, msg_ref, out_ref):
    i = pl.program_id(0)
    colid = i * BLK + lax.broadcasted_iota(jnp.int32, (M, BLK), 1)
    wt = jnp.zeros((M, BLK), jnp.float32)
    for k in range(K):
        hit = (tn_ref[:, k][:, None] == colid).astype(jnp.float32)
        wt = wt + attn_ref[:, k][:, None] * hit
    out_ref[...] = lax.dot_general(wt, msg_ref[...], (((0,), (0,)), ((), ())),
                                   preferred_element_type=jnp.float32)


def _contrib(tn, attn_pad, msg):
    return pl.pallas_call(
        _contrib_body,
        grid=(NBLK,),
        in_specs=[pl.BlockSpec((M, K), lambda i: (0, 0)),
                  pl.BlockSpec((M, D), lambda i: (0, 0)),
                  pl.BlockSpec((M, D), lambda i: (0, 0))],
        out_specs=pl.BlockSpec((BLK, D), lambda i: (i, 0)),
        out_shape=jax.ShapeDtypeStruct((N, D), jnp.float32),
    )(tn, attn_pad, msg)


def _lnbn_body(x_ref, g_ref, b_ref, y_ref, s1_ref, s2_ref):
    x = x_ref[...]
    mu = jnp.mean(x, -1, keepdims=True)
    v = jnp.mean(x * x, -1, keepdims=True) - mu * mu
    y = _leaky(g_ref[0:1, :] * (x - mu) * lax.rsqrt(v + 1e-5) + b_ref[0:1, :])
    y_ref[...] = y
    row0 = lax.broadcasted_iota(jnp.int32, (8, D), 0) == 0
    s1_ref[...] = jnp.where(row0, jnp.sum(y, axis=0, keepdims=True), 0.0)
    s2_ref[...] = jnp.where(row0, jnp.sum(y * y, axis=0, keepdims=True), 0.0)


def _lnbn_pass(x, g8, b8):
    return pl.pallas_call(
        _lnbn_body,
        grid=(NBLK,),
        in_specs=[pl.BlockSpec((BLK, D), lambda i: (i, 0)),
                  pl.BlockSpec((8, D), lambda i: (0, 0)),
                  pl.BlockSpec((8, D), lambda i: (0, 0))],
        out_specs=[pl.BlockSpec((BLK, D), lambda i: (i, 0)),
                   pl.BlockSpec((8, D), lambda i: (i, 0)),
                   pl.BlockSpec((8, D), lambda i: (i, 0))],
        out_shape=(jax.ShapeDtypeStruct((N, D), jnp.float32),
                   jax.ShapeDtypeStruct((NBLK * 8, D), jnp.float32),
                   jax.ShapeDtypeStruct((NBLK * 8, D), jnp.float32)),
    )(x, g8, b8)


def _dense_gat(h, ssrc, sdst, src, dst, n):
    oh_dst_n = (lax.broadcasted_iota(jnp.int32, (n, EMG), 0)
                == dst[None, :]).astype(jnp.float32)
    oh_src_e = (lax.broadcasted_iota(jnp.int32, (EMG, n), 1)
                == src[:, None]).astype(jnp.float32)
    oh_dst_e = (lax.broadcasted_iota(jnp.int32, (EMG, n), 1)
                == dst[:, None]).astype(jnp.float32)
    sc_src = jnp.dot(oh_src_e, ssrc, preferred_element_type=jnp.float32)
    sc_dst = jnp.dot(oh_dst_e, sdst, preferred_element_type=jnp.float32)
    e = _leaky(sc_src + sc_dst)
    big = jnp.float32(-1e30)
    m = jnp.max(jnp.where(oh_dst_n[:, :, None] > 0, e[None, :, :], big), axis=1)
    m = jnp.where(m <= big * 0.5, 0.0, m)
    ex = jnp.exp(e - jnp.dot(oh_dst_e, m, preferred_element_type=jnp.float32))
    den = jnp.dot(oh_dst_n, ex, preferred_element_type=jnp.float32)
    alpha = ex / (jnp.dot(oh_dst_e, den, preferred_element_type=jnp.float32) + 1e-16)
    hsrc = jnp.dot(oh_src_e, h, preferred_element_type=jnp.float32)
    wrow = (alpha[:, :, None] * hsrc.reshape(EMG, HEADS, HID)).reshape(EMG, D)
    return jnp.dot(oh_dst_n, wrow, preferred_element_type=jnp.float32)


def _mg_body(do_norm, *refs):
    (mgx_ref, nb_ref, mp_ref, ei_ref,
     wm_ref, bsm_ref, bdm_ref, wsem_ref, bq_ref, qv_ref,
     wpc_ref, vpc_ref, wpool_ref, wme_ref, bsme_ref, bdme_ref, wcci_ref,
     lng_ref, lnb_ref, bng_ref, bnb_ref,
     mg_out_ref, attn_out_ref, msg_ref) = refs
    x = mgx_ref[...]
    if do_norm:
        mu = jnp.mean(x, -1, keepdims=True)
        v = jnp.mean(x * x, -1, keepdims=True) - mu * mu
        x = _leaky(lng_ref[0:1, :] * (x - mu) * lax.rsqrt(v + 1e-5) + lnb_ref[0:1, :])
        mu2 = jnp.mean(x, 0, keepdims=True)
        v2 = jnp.mean(x * x, 0, keepdims=True) - mu2 * mu2
        x = bng_ref[0:1, :] * (x - mu2) * lax.rsqrt(v2 + 1e-5) + bnb_ref[0:1, :]

    mp = mp_ref[...].reshape(RREL * 2 * EMG)
    ei = ei_ref[...].reshape(2 * EMG)

    zs = []
    ws = []
    for r in range(RREL):
        w = wm_ref[...][r * D:(r + 1) * D, :]
        h = jnp.dot(x, w, preferred_element_type=jnp.float32)
        ssrc = jnp.dot(h, bsm_ref[...][r * D:(r + 1) * D, :],
                       preferred_element_type=jnp.float32)
        sdst = jnp.dot(h, bdm_ref[...][r * D:(r + 1) * D, :],
                       preferred_element_type=jnp.float32)
        src = mp[r * 2 * EMG:r * 2 * EMG + EMG]
        dst = mp[r * 2 * EMG + EMG:(r + 1) * 2 * EMG]
        z = _dense_gat(h, ssrc, sdst, src, dst, M)
        zs.append(z)
        t = jnp.tanh(jnp.dot(z, wsem_ref[...],
                             preferred_element_type=jnp.float32) + bq_ref[0:1, :])
        ws.append(jnp.mean(jnp.sum(t * qv_ref[0:1, :], axis=1)))
    w0 = ws[0]; w1 = ws[1]
    mx = jnp.maximum(w0, w1)
    e0 = jnp.exp(w0 - mx); e1 = jnp.exp(w1 - mx)
    b0 = e0 / (e0 + e1); b1 = e1 / (e0 + e1)
    mg_sem = b0 * zs[0] + b1 * zs[1]

    nb = nb_ref[...]
    t = jnp.tanh(jnp.dot(nb, wpc_ref[...], preferred_element_type=jnp.float32))
    y = jnp.sum(t * vpc_ref[0:1, :], axis=1).reshape(M, K)
    ymax = jnp.max(y, axis=1, keepdims=True)
    yex = jnp.exp(y - ymax)
    attn = yex / jnp.sum(yex, axis=1, keepdims=True)
    pooled = jnp.sum(attn[:, :, None] * nb.reshape(M, K, D), axis=1)

    he = jnp.dot(x, wme_ref[...], preferred_element_type=jnp.float32)
    ssrc = jnp.dot(he, bsme_ref[...], preferred_element_type=jnp.float32)
    sdst = jnp.dot(he, bdme_ref[...], preferred_element_type=jnp.float32)
    src = ei[0:EMG]
    dst = ei[EMG:2 * EMG]
    ge = _dense_gat(he, ssrc, sdst, src, dst, M)

    mg = _leaky(mg_sem + jnp.dot(pooled, wpool_ref[...],
                                 preferred_element_type=jnp.float32) + ge)
    mg_out_ref[...] = mg
    attn_out_ref[...] = jnp.concatenate(
        [attn, jnp.zeros((M, D - K), jnp.float32)], axis=1)
    msg_ref[...] = jnp.dot(mg, wcci_ref[...], preferred_element_type=jnp.float32)


def _mg_all(do_norm, mgx, nb, mp_i, ei_i, wm, bsm, bdm, wsem, bq, qv,
            wpc, vpc, wpool, wme, bsme, bdme, wcci, lng, lnb, bng, bnb):
    full = lambda s: pl.BlockSpec(s, lambda: tuple(0 for _ in s))
    ins = [mgx, nb, mp_i, ei_i, wm, bsm, bdm, wsem, bq, qv,
           wpc, vpc, wpool, wme, bsme, bdme, wcci, lng, lnb, bng, bnb]
    in_specs = [full(tuple(a.shape)) for a in ins]
    return pl.pallas_call(
        functools.partial(_mg_body, do_norm),
        in_specs=in_specs,
        out_specs=[full((M, D)), full((M, D)), full((M, D))],
        out_shape=(jax.ShapeDtypeStruct((M, D), jnp.float32),
                   jax.ShapeDtypeStruct((M, D), jnp.float32),
                   jax.ShapeDtypeStruct((M, D), jnp.float32)),
    )(*ins)


def _blockdiag(a):
    eye = jnp.eye(HEADS, dtype=a.dtype)
    return (eye[:, None, :] * a[:, :, None]).reshape(D, HEADS)


def _pad8(v):
    return jnp.broadcast_to(v[None, :], (8, v.shape[0]))


def _beta_from_partials(wp, n):
    t = jnp.sum(wp, axis=0)
    w = t[:2] / n
    return jax.nn.softmax(w)


def _up_layer(x, mgx, mp_srcdst, ei_srcdst, mg_mp_i, mg_ei_i, tn_flat, p,
              wcci, scale=None, shift=None, do_mg_norm=False, ln=None):
    wcat = jnp.concatenate([p['Wp'][0], p['Wp'][1], p['Wpe']], 1)
    a_s = jnp.concatenate([p['asp'][0], p['asp'][1], p['aspe']], 0)
    a_d = jnp.concatenate([p['adp'][0], p['adp'][1], p['adpe']], 0)
    hs0, sd0, hs1, sd1, hs2, sd2 = _mm_proj(x, wcat, a_s, a_d, 3,
                                            scale=scale, shift=shift)

    acc0 = _sc_gat(hs0, sd0, mp_srcdst[0][0], mp_srcdst[0][1])
    acc1 = _sc_gat(hs1, sd1, mp_srcdst[1][0], mp_srcdst[1][1])
    acc2 = _sc_gat(hs2, sd2, ei_srcdst[0], ei_srcdst[1])

    z0, z1, ze, wp = _post([acc0, acc1], acc2,
                           p['Wsem_p'], _pad8(p['bsem_p']), _pad8(p['qsem_p']))
    beta = _beta_from_partials(wp, N)
    ppi_sem, ppi_out = _combine(z0, z1, ze, beta)

    nb = _sc_gather(ppi_sem, tn_flat)
    lng, lnb, bng, bnb = ln
    mg_out, attn_pad, msg = _mg_all(
        do_mg_norm, mgx, nb, mg_mp_i, mg_ei_i,
        p['Wm'].reshape(RREL * D, D),
        jnp.concatenate([_blockdiag(p['asm'][r]) for r in range(RREL)], 0),
        jnp.concatenate([_blockdiag(p['adm'][r]) for r in range(RREL)], 0),
        p['Wsem_m'], _pad8(p['bsem_m']), _pad8(p['qsem_m']),
        p['Wpc'], _pad8(p['vpc']), p['Wpool'],
        p['Wme'], _blockdiag(p['asme']), _blockdiag(p['adme']),
        wcci,
        _pad8(lng), _pad8(lnb), _pad8(bng), _pad8(bnb))
    return ppi_out, mg_out, attn_pad, msg


def _down_layer(x, msg, attn_pad, tn, mp_srcdst, p):
    wcat = jnp.concatenate([p['Wd'][0], p['Wd'][1]], 1)
    a_s = jnp.concatenate([p['asd'][0], p['asd'][1]], 0)
    a_d = jnp.concatenate([p['add'][0], p['add'][1]], 0)
    hs0, sd0, hs1, sd1 = _mm_proj(x, wcat, a_s, a_d, 2)

    acc0 = _sc_gat(hs0, sd0, mp_srcdst[0][0], mp_srcdst[0][1])
    acc1 = _sc_gat(hs1, sd1, mp_srcdst[1][0], mp_srcdst[1][1])

    z0, z1, wp = _post([acc0, acc1], None,
                       p['Wsem_d'], _pad8(p['bsem_d']), _pad8(p['qsem_d']))
    beta = _beta_from_partials(wp, N)
    contrib = _contrib(tn, attn_pad, msg)
    _, out = _combine(z0, z1, contrib, beta)
    return out


def kernel(ppi_x, mg_x, ppi_metapaths, mg_metapaths, ppi_edge_index,
           mg_edge_index, tissue_neighbors, params):
    mp = [(ppi_metapaths[r, 0], ppi_metapaths[r, 1]) for r in range(RREL)]
    ei = (ppi_edge_index[0], ppi_edge_index[1])
    mg_mp_i = mg_metapaths.reshape(RREL * 2 * EMG // D, D)
    mg_ei_i = mg_edge_index.reshape(2 * EMG // D, D)
    tn_flat = tissue_neighbors.reshape(M * K)
    ln = (params['ln_g'], params['ln_b'], params['bn_g'], params['bn_b'])

    p1u, p1d = params['conv1_up'], params['conv1_down']
    p2u, p2d = params['conv2_up'], params['conv2_down']

    ppi1, mg1, attn1, msg1 = _up_layer(ppi_x, mg_x, mp, ei, mg_mp_i, mg_ei_i,
                                       tn_flat, p1u, p1d['Wcci'], ln=ln)
    ppi2 = _down_layer(ppi1, msg1, attn1, tissue_neighbors, mp, p1d)

    y, s1, s2 = _lnbn_pass(ppi2, _pad8(params['ln_g']), _pad8(params['ln_b']))
    mu = jnp.sum(s1, 0) / N
    var = jnp.sum(s2, 0) / N - mu * mu
    scale = params['bn_g'] * lax.rsqrt(var + 1e-5)
    shift = params['bn_b'] - mu * scale

    ppi3, mg2, attn2, msg2 = _up_layer(y, mg1, mp, ei, mg_mp_i, mg_ei_i,
                                       tn_flat, p2u, p2d['Wcci'],
                                       scale=scale, shift=shift,
                                       do_mg_norm=True, ln=ln)
    ppi4 = _down_layer(ppi3, msg2, attn2, tissue_neighbors, mp, p2d)
    return ppi4, mg2
